# Initial kernel scaffold; baseline (speedup 1.0000x reference)
import jax
import jax.numpy as jnp
from jax import lax
from jax.experimental import pallas as pl
from jax.experimental.pallas import tpu as pltpu
from jax.experimental.pallas import tpu_sc as plsc

N = 10000
E = 320000
D = 128
HID = 64
OUT = 128
NH = 2

NC, NS, L = 2, 16, 16
NW = NC * NS
NPAD = 10112
NBLK = NPAD // 128

_SELU_L = 1.0507009873554805
_SELU_A = 1.6732632423543772


def _mesh():
    return plsc.VectorSubcoreMesh(core_axis_name="c", subcore_axis_name="s",
                                  num_cores=NC, num_subcores=NS)


def _splat(v, j):
    return jnp.take(v, jnp.full((L,), j, jnp.int32), mode="promise_in_bounds")


def _zero_ref_1d(ref, n):
    z = jnp.zeros((L,), jnp.float32)

    def body(i, _):
        ref[pl.ds(i * L, L)] = z
        return 0

    lax.fori_loop(0, n // L, body, 0)


def _zero_rows(ref, nrows, width):
    z = jnp.zeros((L,), jnp.float32)

    def body(i, _):
        for k in range(width // L):
            ref[i, pl.ds(k * L, L)] = z
        return 0

    lax.fori_loop(0, nrows, body, 0)


def _acc_zero_and_writeback(rows, acc, dst, r0, zero_phase):
    if zero_phase:
        for k in range(4):
            pltpu.sync_copy(rows, acc.at[pl.ds(r0 + k * 128, 128)])
        pltpu.sync_copy(rows.at[pl.ds(0, 120)], acc.at[pl.ds(r0 + 512, 120)])
    else:
        for k in range(4):
            pltpu.sync_copy(acc.at[pl.ds(r0 + k * 128, 128)],
                            dst.at[pl.ds(r0 + k * 128, 128)])
        pltpu.sync_copy(acc.at[pl.ds(r0 + 512, 120)],
                        dst.at[pl.ds(r0 + 512, 120)])


def _deg_sc(col_hbm, ew_hbm, out_hbm, colbuf, ewbuf, hist):
    c = lax.axis_index("c")
    s = lax.axis_index("s")
    wid = c * NS + s
    _zero_ref_1d(hist, NPAD)
    eper = E // NW
    chunk = 2000

    def chunk_body(t, _):
        base = wid * eper + t * chunk
        pltpu.sync_copy(col_hbm.at[pl.ds(base, chunk)], colbuf)
        pltpu.sync_copy(ew_hbm.at[pl.ds(base, chunk)], ewbuf)

        def grp(g, _):
            cv = colbuf[pl.ds(g * L, L)]
            ev = ewbuf[pl.ds(g * L, L)]
            plsc.addupdate_scatter(hist, [cv], ev)
            return 0

        lax.fori_loop(0, chunk // L, grp, 0)
        return 0

    lax.fori_loop(0, eper // chunk, chunk_body, 0)
    pltpu.sync_copy(hist, out_hbm.at[wid])


def _deg_kernel(col, ew):
    return pl.kernel(
        _deg_sc,
        out_type=jax.ShapeDtypeStruct((NW, NPAD), jnp.float32),
        mesh=_mesh(),
        scratch_types=[
            pltpu.VMEM((2000,), jnp.int32),
            pltpu.VMEM((2000,), jnp.float32),
            pltpu.VMEM((NPAD,), jnp.float32),
        ],
    )(col, ew)


def _prep_tc(hist_ref, x_ref, dinv_ref, dsq_ref, g_ref):
    deg = jnp.sum(hist_ref[...], axis=0) + 1.0
    dinv = lax.rsqrt(deg)
    dinv_ref[...] = dinv
    dsq_ref[...] = 1.0 / deg
    g_ref[...] = x_ref[...] * dinv


def _prep_kernel(hists, xpad):
    return pl.pallas_call(
        _prep_tc,
        grid=(NBLK,),
        in_specs=[
            pl.BlockSpec((NW, 128, 1), lambda i: (0, i, 0)),
            pl.BlockSpec((128, D), lambda i: (i, 0)),
        ],
        out_specs=[
            pl.BlockSpec((128, 1), lambda i: (i, 0)),
            pl.BlockSpec((128, 1), lambda i: (i, 0)),
            pl.BlockSpec((128, D), lambda i: (i, 0)),
        ],
        out_shape=[
            jax.ShapeDtypeStruct((NPAD, 1), jnp.float32),
            jax.ShapeDtypeStruct((NPAD, 1), jnp.float32),
            jax.ShapeDtypeStruct((NPAD, D), jnp.float32),
        ],
    )(hists.reshape(NW, NPAD, 1), xpad)


def _scale_rows(rows, wbuf, nrows):
    def grp(g, _):
        wv = wbuf[pl.ds(g * L, L)]
        for j in range(L):
            cj = _splat(wv, j)
            for k in range(D // L):
                sl = pl.ds(k * L, L)
                rows[g * L + j, sl] = rows[g * L + j, sl] * cj
        return 0

    lax.fori_loop(0, nrows // L, grp, 0)


def _hop_sc(g_hbm, row_hbm, col_hbm, ew_hbm, out_hbm,
            rowbuf, colbuf, ewbuf, rows, rowbufE, colbufE, ewbufE, rowsE,
            acc, sem):
    c = lax.axis_index("c")
    s = lax.axis_index("s")
    wid = c * NS + s
    eper = E // NW
    r0 = s * (NPAD // NS)

    _zero_rows(rows, 128, D)
    _acc_zero_and_writeback(rows, acc, None, r0, True)
    plsc.subcore_barrier()

    def chunk_body(t, _):
        base = wid * eper + t * 128
        pltpu.sync_copy(row_hbm.at[pl.ds(base, 128)], rowbuf)
        pltpu.sync_copy(col_hbm.at[pl.ds(base, 128)], colbuf)
        pltpu.sync_copy(ew_hbm.at[pl.ds(base, 128)], ewbuf)
        pltpu.async_copy(g_hbm.at[rowbuf], rows, sem).wait()
        _scale_rows(rows, ewbuf, 128)
        pltpu.sync_copy(rows, acc.at[colbuf], add=True)
        return 0

    lax.fori_loop(0, 78, chunk_body, 0)

    base = wid * eper + 78 * 128
    pltpu.sync_copy(row_hbm.at[pl.ds(base, 16)], rowbufE)
    pltpu.sync_copy(col_hbm.at[pl.ds(base, 16)], colbufE)
    pltpu.sync_copy(ew_hbm.at[pl.ds(base, 16)], ewbufE)
    pltpu.async_copy(g_hbm.at[rowbufE], rowsE, sem).wait()
    _scale_rows(rowsE, ewbufE, 16)
    pltpu.sync_copy(rowsE, acc.at[colbufE], add=True)

    plsc.subcore_barrier()
    _acc_zero_and_writeback(rows, acc, out_hbm.at[c], r0, False)


def _hop_kernel(g, row, col, ew):
    return pl.kernel(
        _hop_sc,
        out_type=jax.ShapeDtypeStruct((NC, NPAD, D), jnp.float32),
        mesh=_mesh(),
        scratch_types=[
            pltpu.VMEM((128,), jnp.int32),
            pltpu.VMEM((128,), jnp.int32),
            pltpu.VMEM((128,), jnp.float32),
            pltpu.VMEM((128, D), jnp.float32),
            pltpu.VMEM((16,), jnp.int32),
            pltpu.VMEM((16,), jnp.int32),
            pltpu.VMEM((16,), jnp.float32),
            pltpu.VMEM((16, D), jnp.float32),
            pltpu.VMEM_SHARED((NPAD, D), jnp.float32),
            pltpu.SemaphoreType.DMA,
        ],
    )(g, row, col, ew)


def _combine_tc(p_ref, dinv_ref, dsq_ref, h_ref, hn_ref, gn_ref):
    sm = p_ref[0] + p_ref[1]
    hn = dinv_ref[...] * sm + dsq_ref[...] * h_ref[...]
    hn_ref[...] = hn
    gn_ref[...] = dinv_ref[...] * hn


def _combine_kernel(p, dinv, dsq, h):
    return pl.pallas_call(
        _combine_tc,
        grid=(NBLK,),
        in_specs=[
            pl.BlockSpec((NC, 128, D), lambda i: (0, i, 0)),
            pl.BlockSpec((128, 1), lambda i: (i, 0)),
            pl.BlockSpec((128, 1), lambda i: (i, 0)),
            pl.BlockSpec((128, D), lambda i: (i, 0)),
        ],
        out_specs=[
            pl.BlockSpec((128, D), lambda i: (i, 0)),
            pl.BlockSpec((128, D), lambda i: (i, 0)),
        ],
        out_shape=[
            jax.ShapeDtypeStruct((NPAD, D), jnp.float32),
            jax.ShapeDtypeStruct((NPAD, D), jnp.float32),
        ],
    )(p, dinv, dsq, h)


def _mm_tc(h_ref, wsg_ref, bsg_ref, wgat_ref, asrc_ref, adst_ref,
           xh_ref, a4_ref):
    t = jnp.dot(h_ref[...], wsg_ref[...],
                preferred_element_type=jnp.float32) + bsg_ref[...]
    t = _SELU_L * jnp.where(t > 0, t, _SELU_A * jnp.expm1(t))
    y = jnp.dot(t, wgat_ref[...], preferred_element_type=jnp.float32)
    y0 = y[:, :OUT]
    y1 = y[:, OUT:]
    xh_ref[0] = y0
    xh_ref[1] = y1
    a4_ref[...] = jnp.concatenate([
        jnp.sum(y0 * asrc_ref[0:1, :], axis=1, keepdims=True),
        jnp.sum(y1 * asrc_ref[1:2, :], axis=1, keepdims=True),
        jnp.sum(y0 * adst_ref[0:1, :], axis=1, keepdims=True),
        jnp.sum(y1 * adst_ref[1:2, :], axis=1, keepdims=True),
    ], axis=1)


def _mm_kernel(h2, W_sg, b_sg, W_gat, att_src, att_dst):
    return pl.pallas_call(
        _mm_tc,
        grid=(NBLK,),
        in_specs=[
            pl.BlockSpec((128, D), lambda i: (i, 0)),
            pl.BlockSpec((D, HID), lambda i: (0, 0)),
            pl.BlockSpec((1, HID), lambda i: (0, 0)),
            pl.BlockSpec((HID, NH * OUT), lambda i: (0, 0)),
            pl.BlockSpec((NH, OUT), lambda i: (0, 0)),
            pl.BlockSpec((NH, OUT), lambda i: (0, 0)),
        ],
        out_specs=[
            pl.BlockSpec((NH, 128, OUT), lambda i: (0, i, 0)),
            pl.BlockSpec((128, 4), lambda i: (i, 0)),
        ],
        out_shape=[
            jax.ShapeDtypeStruct((NH, NPAD, OUT), jnp.float32),
            jax.ShapeDtypeStruct((NPAD, 4), jnp.float32),
        ],
    )(h2, W_sg, b_sg.reshape(1, HID), W_gat, att_src, att_dst)


def _gat_sc(xh_hbm, row_hbm, col_hbm, asrc_hbm, adst_hbm,
            o_hbm, ah_hbm,
            rowbuf, colbuf, rows, rowbufE, colbufE, rowsE,
            asrcv, adstv, abuf, hist, acc, sem):
    c = lax.axis_index("c")
    s = lax.axis_index("s")
    eper = E // NS
    cbase = c * NPAD
    r0 = s * (NPAD // NS)

    pltpu.sync_copy(asrc_hbm.at[c], asrcv)
    pltpu.sync_copy(adst_hbm.at[c], adstv)
    _zero_ref_1d(hist, NPAD)
    _zero_rows(rows, 128, OUT)
    _acc_zero_and_writeback(rows, acc, None, r0, True)
    plsc.subcore_barrier()

    def do_chunk(base, rb, cb, rw, nrows):
        pltpu.sync_copy(row_hbm.at[pl.ds(base, nrows)], rb)
        pltpu.sync_copy(col_hbm.at[pl.ds(base, nrows)], cb)

        def grp(g, _):
            sl = pl.ds(g * L, L)
            rv = rb[sl]
            cv = cb[sl]
            sv = plsc.load_gather(asrcv, [rv])
            dv = plsc.load_gather(adstv, [cv])
            t = sv + dv
            t = jnp.where(t >= 0, t, 0.2 * t)
            al = jnp.exp(t)
            abuf[sl] = al
            plsc.addupdate_scatter(hist, [cv], al)
            rb[sl] = rv + cbase
            return 0

        lax.fori_loop(0, nrows // L, grp, 0)
        pltpu.async_copy(xh_hbm.at[rb], rw, sem).wait()
        _scale_rows(rw, abuf, nrows)
        pltpu.sync_copy(rw, acc.at[cb], add=True)

    def chunk_body(t, _):
        do_chunk(s * eper + t * 128, rowbuf, colbuf, rows, 128)
        return 0

    lax.fori_loop(0, 156, chunk_body, 0)
    do_chunk(s * eper + 156 * 128, rowbufE, colbufE, rowsE, 32)

    pltpu.sync_copy(hist, ah_hbm.at[c, s])
    plsc.subcore_barrier()
    _acc_zero_and_writeback(rows, acc, o_hbm.at[c], r0, False)


def _gat_kernel(xhflat, row, col, asrc2, adst2):
    return pl.kernel(
        _gat_sc,
        out_type=[
            jax.ShapeDtypeStruct((NC, NPAD, OUT), jnp.float32),
            jax.ShapeDtypeStruct((NC, NS, NPAD), jnp.float32),
        ],
        mesh=_mesh(),
        scratch_types=[
            pltpu.VMEM((128,), jnp.int32),
            pltpu.VMEM((128,), jnp.int32),
            pltpu.VMEM((128, OUT), jnp.float32),
            pltpu.VMEM((32,), jnp.int32),
            pltpu.VMEM((32,), jnp.int32),
            pltpu.VMEM((32, OUT), jnp.float32),
            pltpu.VMEM((NPAD,), jnp.float32),
            pltpu.VMEM((NPAD,), jnp.float32),
            pltpu.VMEM((128,), jnp.float32),
            pltpu.VMEM((NPAD,), jnp.float32),
            pltpu.VMEM_SHARED((NPAD, OUT), jnp.float32),
            pltpu.SemaphoreType.DMA,
        ],
    )(xhflat, row, col, asrc2, adst2)


def _final_tc(o_ref, ah_ref, a4_ref, xh_ref, bg_ref, out_ref):
    a4 = a4_ref[...]
    t0 = a4[:, 0:1] + a4[:, 2:3]
    t1 = a4[:, 1:2] + a4[:, 3:4]
    aL0 = jnp.exp(jnp.where(t0 >= 0, t0, 0.2 * t0))
    aL1 = jnp.exp(jnp.where(t1 >= 0, t1, 0.2 * t1))
    den0 = jnp.sum(ah_ref[0], axis=0) + aL0 + 1e-16
    den1 = jnp.sum(ah_ref[1], axis=0) + aL1 + 1e-16
    num0 = o_ref[0] + aL0 * xh_ref[0]
    num1 = o_ref[1] + aL1 * xh_ref[1]
    out_ref[...] = 0.5 * (num0 / den0 + num1 / den1) + bg_ref[...]


def _final_kernel(o, ah, a4, xh, b_gat):
    return pl.pallas_call(
        _final_tc,
        grid=(NBLK,),
        in_specs=[
            pl.BlockSpec((NC, 128, OUT), lambda i: (0, i, 0)),
            pl.BlockSpec((NC, NS, 128, 1), lambda i: (0, 0, i, 0)),
            pl.BlockSpec((128, 4), lambda i: (i, 0)),
            pl.BlockSpec((NC, 128, OUT), lambda i: (0, i, 0)),
            pl.BlockSpec((1, OUT), lambda i: (0, 0)),
        ],
        out_specs=pl.BlockSpec((128, OUT), lambda i: (i, 0)),
        out_shape=jax.ShapeDtypeStruct((NPAD, OUT), jnp.float32),
    )(o, ah.reshape(NC, NS, NPAD, 1), a4, xh, b_gat.reshape(1, OUT))


def kernel(x, edge_index, edge_attr, W_sg, b_sg, W_gat, att_src, att_dst,
           b_gat):
    row = edge_index[0]
    col = edge_index[1]
    xpad = jnp.pad(x, ((0, NPAD - N), (0, 0)))

    hists = _deg_kernel(col, edge_attr)
    dinv, dsq, g = _prep_kernel(hists, xpad)
    p1 = _hop_kernel(g, row, col, edge_attr)
    h1, g1 = _combine_kernel(p1, dinv, dsq, xpad)
    p2 = _hop_kernel(g1, row, col, edge_attr)
    h2, _ = _combine_kernel(p2, dinv, dsq, h1)

    xh, a4 = _mm_kernel(h2, W_sg, b_sg, W_gat, att_src, att_dst)
    asrc2 = jnp.transpose(a4[:, 0:2])
    adst2 = jnp.transpose(a4[:, 2:4])
    o, ah = _gat_kernel(xh.reshape(NC * NPAD, OUT), row, col, asrc2, adst2)
    out = _final_kernel(o, ah, a4, xh, b_gat)
    return out[:N]


# speedup vs baseline: 27.9949x; 27.9949x over previous
import jax
import jax.numpy as jnp
from jax import lax
from jax.experimental import pallas as pl
from jax.experimental.pallas import tpu as pltpu
from jax.experimental.pallas import tpu_sc as plsc

N = 10000
E = 320000
D = 128
HID = 64
OUT = 128
NH = 2

NC, NS, L = 2, 16, 16
NW = NC * NS
NPAD = 10112
NBLK = NPAD // 128

_SELU_L = 1.0507009873554805
_SELU_A = 1.6732632423543772


def _mesh():
    return plsc.VectorSubcoreMesh(core_axis_name="c", subcore_axis_name="s",
                                  num_cores=NC, num_subcores=NS)


def _splat(v, j):
    return lax.gather(
        v, jnp.full((L, 1), j, jnp.int32),
        dimension_numbers=lax.GatherDimensionNumbers(
            offset_dims=(), collapsed_slice_dims=(0,), start_index_map=(0,)),
        slice_sizes=(1,), mode=lax.GatherScatterMode.PROMISE_IN_BOUNDS)


def _zero_ref_1d(ref, n):
    z = jnp.zeros((L,), jnp.float32)

    def body(i, _):
        ref[pl.ds(i * L, L)] = z
        return 0

    lax.fori_loop(0, n // L, body, 0)


def _zero_rows(ref, nrows, width):
    z = jnp.zeros((L,), jnp.float32)

    def body(i, _):
        for k in range(width // L):
            ref[i, pl.ds(k * L, L)] = z
        return 0

    lax.fori_loop(0, nrows, body, 0)


def _acc_zero_and_writeback(rows, acc, dst, r0, zero_phase):
    if zero_phase:
        for k in range(4):
            pltpu.sync_copy(rows, acc.at[pl.ds(r0 + k * 128, 128)])
        pltpu.sync_copy(rows.at[pl.ds(0, 120)], acc.at[pl.ds(r0 + 512, 120)])
    else:
        for k in range(4):
            pltpu.sync_copy(acc.at[pl.ds(r0 + k * 128, 128)],
                            dst.at[pl.ds(r0 + k * 128, 128)])
        pltpu.sync_copy(acc.at[pl.ds(r0 + 512, 120)],
                        dst.at[pl.ds(r0 + 512, 120)])


def _deg_sc(col_hbm, ew_hbm, out_hbm, colbuf, ewbuf, hist):
    c = lax.axis_index("c")
    s = lax.axis_index("s")
    wid = c * NS + s
    _zero_ref_1d(hist, NPAD)
    eper = E // NW
    chunk = 2000

    def chunk_body(t, _):
        base = wid * eper + t * chunk
        pltpu.sync_copy(col_hbm.at[pl.ds(base, chunk)], colbuf)
        pltpu.sync_copy(ew_hbm.at[pl.ds(base, chunk)], ewbuf)

        def grp(g, _):
            cv = colbuf[pl.ds(g * L, L)]
            ev = ewbuf[pl.ds(g * L, L)]
            plsc.addupdate_scatter(hist, [cv], ev)
            return 0

        lax.fori_loop(0, chunk // L, grp, 0)
        return 0

    lax.fori_loop(0, eper // chunk, chunk_body, 0)
    pltpu.sync_copy(hist, out_hbm.at[wid])


def _deg_kernel(col, ew):
    return pl.kernel(
        _deg_sc,
        out_type=jax.ShapeDtypeStruct((NW, NPAD), jnp.float32),
        mesh=_mesh(),
        compiler_params=pltpu.CompilerParams(needs_layout_passes=False),
        scratch_types=[
            pltpu.VMEM((2000,), jnp.int32),
            pltpu.VMEM((2000,), jnp.float32),
            pltpu.VMEM((NPAD,), jnp.float32),
        ],
    )(col, ew)


def _prep_tc(hist_ref, x_ref, dinv_ref, dsq_ref, g_ref):
    deg = jnp.sum(hist_ref[...], axis=0) + 1.0
    dinv = lax.rsqrt(deg)
    dinv_ref[...] = dinv
    dsq_ref[...] = 1.0 / deg
    g_ref[...] = x_ref[...] * dinv


def _prep_kernel(hists, xpad):
    return pl.pallas_call(
        _prep_tc,
        grid=(NBLK,),
        in_specs=[
            pl.BlockSpec((NW, 128, 1), lambda i: (0, i, 0)),
            pl.BlockSpec((128, D), lambda i: (i, 0)),
        ],
        out_specs=[
            pl.BlockSpec((128, 1), lambda i: (i, 0)),
            pl.BlockSpec((128, 1), lambda i: (i, 0)),
            pl.BlockSpec((128, D), lambda i: (i, 0)),
        ],
        out_shape=[
            jax.ShapeDtypeStruct((NPAD, 1), jnp.float32),
            jax.ShapeDtypeStruct((NPAD, 1), jnp.float32),
            jax.ShapeDtypeStruct((NPAD, D), jnp.float32),
        ],
    )(hists.reshape(NW, NPAD, 1), xpad)


def _scale_rows(rows, wbuf, nrows):
    def grp(g, _):
        wv = wbuf[pl.ds(g * L, L)]
        for j in range(L):
            cj = _splat(wv, j)
            for k in range(D // L):
                sl = pl.ds(k * L, L)
                rows[g * L + j, sl] = rows[g * L + j, sl] * cj
        return 0

    lax.fori_loop(0, nrows // L, grp, 0)


def _hop_sc(g_hbm, row_hbm, col_hbm, ew_hbm, out_hbm,
            rowbuf, colbuf, ewbuf, rows, rowbufE, colbufE, ewbufE, rowsE,
            acc, sem):
    c = lax.axis_index("c")
    s = lax.axis_index("s")
    wid = c * NS + s
    eper = E // NW
    r0 = s * (NPAD // NS)

    _zero_rows(rows, 128, D)
    _acc_zero_and_writeback(rows, acc, None, r0, True)
    plsc.subcore_barrier()

    def chunk_body(t, _):
        base = wid * eper + t * 128
        pltpu.sync_copy(row_hbm.at[pl.ds(base, 128)], rowbuf)
        pltpu.sync_copy(col_hbm.at[pl.ds(base, 128)], colbuf)
        pltpu.sync_copy(ew_hbm.at[pl.ds(base, 128)], ewbuf)
        pltpu.async_copy(g_hbm.at[rowbuf], rows, sem).wait()
        _scale_rows(rows, ewbuf, 128)
        pltpu.sync_copy(rows, acc.at[colbuf], add=True)
        return 0

    lax.fori_loop(0, 78, chunk_body, 0)

    base = wid * eper + 78 * 128
    pltpu.sync_copy(row_hbm.at[pl.ds(base, 16)], rowbufE)
    pltpu.sync_copy(col_hbm.at[pl.ds(base, 16)], colbufE)
    pltpu.sync_copy(ew_hbm.at[pl.ds(base, 16)], ewbufE)
    pltpu.async_copy(g_hbm.at[rowbufE], rowsE, sem).wait()
    _scale_rows(rowsE, ewbufE, 16)
    pltpu.sync_copy(rowsE, acc.at[colbufE], add=True)

    plsc.subcore_barrier()
    _acc_zero_and_writeback(rows, acc, out_hbm.at[c], r0, False)


def _hop_kernel(g, row, col, ew):
    return pl.kernel(
        _hop_sc,
        out_type=jax.ShapeDtypeStruct((NC, NPAD, D), jnp.float32),
        mesh=_mesh(),
        compiler_params=pltpu.CompilerParams(needs_layout_passes=False),
        scratch_types=[
            pltpu.VMEM((128,), jnp.int32),
            pltpu.VMEM((128,), jnp.int32),
            pltpu.VMEM((128,), jnp.float32),
            pltpu.VMEM((128, D), jnp.float32),
            pltpu.VMEM((16,), jnp.int32),
            pltpu.VMEM((16,), jnp.int32),
            pltpu.VMEM((16,), jnp.float32),
            pltpu.VMEM((16, D), jnp.float32),
            pltpu.VMEM_SHARED((NPAD, D), jnp.float32),
            pltpu.SemaphoreType.DMA,
        ],
    )(g, row, col, ew)


def _combine_tc(p_ref, dinv_ref, dsq_ref, h_ref, hn_ref, gn_ref):
    sm = p_ref[0] + p_ref[1]
    hn = dinv_ref[...] * sm + dsq_ref[...] * h_ref[...]
    hn_ref[...] = hn
    gn_ref[...] = dinv_ref[...] * hn


def _combine_kernel(p, dinv, dsq, h):
    return pl.pallas_call(
        _combine_tc,
        grid=(NBLK,),
        in_specs=[
            pl.BlockSpec((NC, 128, D), lambda i: (0, i, 0)),
            pl.BlockSpec((128, 1), lambda i: (i, 0)),
            pl.BlockSpec((128, 1), lambda i: (i, 0)),
            pl.BlockSpec((128, D), lambda i: (i, 0)),
        ],
        out_specs=[
            pl.BlockSpec((128, D), lambda i: (i, 0)),
            pl.BlockSpec((128, D), lambda i: (i, 0)),
        ],
        out_shape=[
            jax.ShapeDtypeStruct((NPAD, D), jnp.float32),
            jax.ShapeDtypeStruct((NPAD, D), jnp.float32),
        ],
    )(p, dinv, dsq, h)


def _mm_tc(h_ref, wsg_ref, bsg_ref, wgat_ref, asrc_ref, adst_ref,
           xh_ref, a4_ref):
    t = jnp.dot(h_ref[...], wsg_ref[...],
                preferred_element_type=jnp.float32) + bsg_ref[...]
    t = _SELU_L * jnp.where(t > 0, t, _SELU_A * (jnp.exp(t) - 1.0))
    y = jnp.dot(t, wgat_ref[...], preferred_element_type=jnp.float32)
    y0 = y[:, :OUT]
    y1 = y[:, OUT:]
    xh_ref[0] = y0
    xh_ref[1] = y1
    a4_ref[...] = jnp.concatenate([
        jnp.sum(y0 * asrc_ref[0:1, :], axis=1, keepdims=True),
        jnp.sum(y1 * asrc_ref[1:2, :], axis=1, keepdims=True),
        jnp.sum(y0 * adst_ref[0:1, :], axis=1, keepdims=True),
        jnp.sum(y1 * adst_ref[1:2, :], axis=1, keepdims=True),
    ], axis=1)


def _mm_kernel(h2, W_sg, b_sg, W_gat, att_src, att_dst):
    return pl.pallas_call(
        _mm_tc,
        grid=(NBLK,),
        in_specs=[
            pl.BlockSpec((128, D), lambda i: (i, 0)),
            pl.BlockSpec((D, HID), lambda i: (0, 0)),
            pl.BlockSpec((1, HID), lambda i: (0, 0)),
            pl.BlockSpec((HID, NH * OUT), lambda i: (0, 0)),
            pl.BlockSpec((NH, OUT), lambda i: (0, 0)),
            pl.BlockSpec((NH, OUT), lambda i: (0, 0)),
        ],
        out_specs=[
            pl.BlockSpec((NH, 128, OUT), lambda i: (0, i, 0)),
            pl.BlockSpec((128, 4), lambda i: (i, 0)),
        ],
        out_shape=[
            jax.ShapeDtypeStruct((NH, NPAD, OUT), jnp.float32),
            jax.ShapeDtypeStruct((NPAD, 4), jnp.float32),
        ],
    )(h2, W_sg, b_sg.reshape(1, HID), W_gat, att_src, att_dst)


def _gat_sc(xh_hbm, row_hbm, col_hbm, asrc_hbm, adst_hbm,
            o_hbm, ah_hbm,
            rowbuf, colbuf, rows, rowbufE, colbufE,
            asrcv, adstv, abuf, hist, acc, sem):
    c = lax.axis_index("c")
    s = lax.axis_index("s")
    eper = E // NS
    cbase = c * NPAD
    r0 = s * (NPAD // NS)

    pltpu.sync_copy(asrc_hbm.at[c], asrcv)
    pltpu.sync_copy(adst_hbm.at[c], adstv)
    _zero_ref_1d(hist, NPAD)
    _zero_rows(rows, 128, OUT)
    _acc_zero_and_writeback(rows, acc, None, r0, True)
    plsc.subcore_barrier()

    def do_chunk(base, rb, cb, nrows):
        rw = rows if nrows == 128 else rows.at[pl.ds(0, nrows)]
        pltpu.sync_copy(row_hbm.at[pl.ds(base, nrows)], rb)
        pltpu.sync_copy(col_hbm.at[pl.ds(base, nrows)], cb)


        def grp(g, _):
            sl = pl.ds(g * L, L)
            rv = rb[sl]
            cv = cb[sl]
            sv = plsc.load_gather(asrcv, [rv])
            dv = plsc.load_gather(adstv, [cv])
            t = sv + dv
            t = jnp.where(t >= 0, t, 0.2 * t)
            al = jnp.exp(t)
            abuf[sl] = al
            plsc.addupdate_scatter(hist, [cv], al)
            rb[sl] = rv + cbase
            return 0

        lax.fori_loop(0, nrows // L, grp, 0)
        pltpu.async_copy(xh_hbm.at[rb], rw, sem).wait()
        _scale_rows(rows, abuf, nrows)
        pltpu.sync_copy(rw, acc.at[cb], add=True)

    def chunk_body(t, _):
        do_chunk(s * eper + t * 128, rowbuf, colbuf, 128)
        return 0

    lax.fori_loop(0, 156, chunk_body, 0)
    do_chunk(s * eper + 156 * 128, rowbufE, colbufE, 32)

    pltpu.sync_copy(hist, ah_hbm.at[c, s])
    plsc.subcore_barrier()
    _acc_zero_and_writeback(rows, acc, o_hbm.at[c], r0, False)


def _gat_kernel(xhflat, row, col, asrc2, adst2):
    return pl.kernel(
        _gat_sc,
        out_type=[
            jax.ShapeDtypeStruct((NC, NPAD, OUT), jnp.float32),
            jax.ShapeDtypeStruct((NC, NS, NPAD), jnp.float32),
        ],
        mesh=_mesh(),
        compiler_params=pltpu.CompilerParams(needs_layout_passes=False),
        scratch_types=[
            pltpu.VMEM((128,), jnp.int32),
            pltpu.VMEM((128,), jnp.int32),
            pltpu.VMEM((128, OUT), jnp.float32),
            pltpu.VMEM((32,), jnp.int32),
            pltpu.VMEM((32,), jnp.int32),
            pltpu.VMEM((NPAD,), jnp.float32),
            pltpu.VMEM((NPAD,), jnp.float32),
            pltpu.VMEM((128,), jnp.float32),
            pltpu.VMEM((NPAD,), jnp.float32),
            pltpu.VMEM_SHARED((NPAD, OUT), jnp.float32),
            pltpu.SemaphoreType.DMA,
        ],
    )(xhflat, row, col, asrc2, adst2)


def _final_tc(o_ref, ah_ref, a4_ref, xh_ref, bg_ref, out_ref):
    a4 = a4_ref[...]
    t0 = a4[:, 0:1] + a4[:, 2:3]
    t1 = a4[:, 1:2] + a4[:, 3:4]
    aL0 = jnp.exp(jnp.where(t0 >= 0, t0, 0.2 * t0))
    aL1 = jnp.exp(jnp.where(t1 >= 0, t1, 0.2 * t1))
    den0 = jnp.sum(ah_ref[0], axis=0) + aL0 + 1e-16
    den1 = jnp.sum(ah_ref[1], axis=0) + aL1 + 1e-16
    num0 = o_ref[0] + aL0 * xh_ref[0]
    num1 = o_ref[1] + aL1 * xh_ref[1]
    out_ref[...] = 0.5 * (num0 / den0 + num1 / den1) + bg_ref[...]


def _final_kernel(o, ah, a4, xh, b_gat):
    return pl.pallas_call(
        _final_tc,
        grid=(NBLK,),
        in_specs=[
            pl.BlockSpec((NC, 128, OUT), lambda i: (0, i, 0)),
            pl.BlockSpec((NC, NS, 128, 1), lambda i: (0, 0, i, 0)),
            pl.BlockSpec((128, 4), lambda i: (i, 0)),
            pl.BlockSpec((NC, 128, OUT), lambda i: (0, i, 0)),
            pl.BlockSpec((1, OUT), lambda i: (0, 0)),
        ],
        out_specs=pl.BlockSpec((128, OUT), lambda i: (i, 0)),
        out_shape=jax.ShapeDtypeStruct((NPAD, OUT), jnp.float32),
    )(o, ah.reshape(NC, NS, NPAD, 1), a4, xh, b_gat.reshape(1, OUT))


def kernel(x, edge_index, edge_attr, W_sg, b_sg, W_gat, att_src, att_dst,
           b_gat):
    row = edge_index[0]
    col = edge_index[1]
    xpad = jnp.pad(x, ((0, NPAD - N), (0, 0)))

    hists = _deg_kernel(col, edge_attr)
    dinv, dsq, g = _prep_kernel(hists, xpad)
    p1 = _hop_kernel(g, row, col, edge_attr)
    h1, g1 = _combine_kernel(p1, dinv, dsq, xpad)
    p2 = _hop_kernel(g1, row, col, edge_attr)
    h2, _ = _combine_kernel(p2, dinv, dsq, h1)

    xh, a4 = _mm_kernel(h2, W_sg, b_sg, W_gat, att_src, att_dst)
    asrc2 = jnp.transpose(a4[:, 0:2])
    adst2 = jnp.transpose(a4[:, 2:4])
    o, ah = _gat_kernel(xh.reshape(NC * NPAD, OUT), row, col, asrc2, adst2)
    out = _final_kernel(o, ah, a4, xh, b_gat)
    return out[:N]


# speedup vs baseline: 41.0620x; 1.4668x over previous
import jax
import jax.numpy as jnp
from jax import lax
from jax.experimental import pallas as pl
from jax.experimental.pallas import tpu as pltpu
from jax.experimental.pallas import tpu_sc as plsc

N = 10000
E = 320000
D = 128
HID = 64
OUT = 128
NH = 2

NC, NS, L = 2, 16, 16
NW = NC * NS
NPAD = 10112
NBLK = NPAD // 128

_SELU_L = 1.0507009873554805
_SELU_A = 1.6732632423543772


def _mesh():
    return plsc.VectorSubcoreMesh(core_axis_name="c", subcore_axis_name="s",
                                  num_cores=NC, num_subcores=NS)


def _splat(v, j):
    return lax.gather(
        v, jnp.full((L, 1), j, jnp.int32),
        dimension_numbers=lax.GatherDimensionNumbers(
            offset_dims=(), collapsed_slice_dims=(0,), start_index_map=(0,)),
        slice_sizes=(1,), mode=lax.GatherScatterMode.PROMISE_IN_BOUNDS)


def _zero_ref_1d(ref, n):
    z = jnp.zeros((L,), jnp.float32)

    def body(i, _):
        ref[pl.ds(i * L, L)] = z
        return 0

    lax.fori_loop(0, n // L, body, 0)


def _zero_rows(ref, nrows, width):
    z = jnp.zeros((L,), jnp.float32)

    def body(i, _):
        for k in range(width // L):
            ref[i, pl.ds(k * L, L)] = z
        return 0

    lax.fori_loop(0, nrows, body, 0)


def _acc_zero_and_writeback(rows, acc, dst, r0, zero_phase):
    if zero_phase:
        for k in range(4):
            pltpu.sync_copy(rows, acc.at[pl.ds(r0 + k * 128, 128)])
        pltpu.sync_copy(rows.at[pl.ds(0, 120)], acc.at[pl.ds(r0 + 512, 120)])
    else:
        for k in range(4):
            pltpu.sync_copy(acc.at[pl.ds(r0 + k * 128, 128)],
                            dst.at[pl.ds(r0 + k * 128, 128)])
        pltpu.sync_copy(acc.at[pl.ds(r0 + 512, 120)],
                        dst.at[pl.ds(r0 + 512, 120)])


def _deg_sc(col_hbm, ew_hbm, out_hbm, colbuf, ewbuf, hist):
    c = lax.axis_index("c")
    s = lax.axis_index("s")
    wid = c * NS + s
    _zero_ref_1d(hist, NPAD)
    eper = E // NW
    chunk = 2000

    def chunk_body(t, _):
        base = wid * eper + t * chunk
        pltpu.sync_copy(col_hbm.at[pl.ds(base, chunk)], colbuf)
        pltpu.sync_copy(ew_hbm.at[pl.ds(base, chunk)], ewbuf)

        def grp(g, _):
            cv = colbuf[pl.ds(g * L, L)]
            ev = ewbuf[pl.ds(g * L, L)]
            plsc.addupdate_scatter(hist, [cv], ev)
            return 0

        lax.fori_loop(0, chunk // L, grp, 0)
        return 0

    lax.fori_loop(0, eper // chunk, chunk_body, 0)
    pltpu.sync_copy(hist, out_hbm.at[wid])


def _deg_kernel(col, ew):
    return pl.kernel(
        _deg_sc,
        out_type=jax.ShapeDtypeStruct((NW, NPAD), jnp.float32),
        mesh=_mesh(),
        compiler_params=pltpu.CompilerParams(needs_layout_passes=False),
        scratch_types=[
            pltpu.VMEM((2000,), jnp.int32),
            pltpu.VMEM((2000,), jnp.float32),
            pltpu.VMEM((NPAD,), jnp.float32),
        ],
    )(col, ew)


def _prep_tc(hist_ref, x_ref, dinv_ref, dsq_ref, g_ref):
    deg = jnp.sum(hist_ref[...], axis=0) + 1.0
    dinv = lax.rsqrt(deg)
    dinv_ref[...] = dinv
    dsq_ref[...] = 1.0 / deg
    g_ref[...] = x_ref[...] * dinv


def _prep_kernel(hists, xpad):
    return pl.pallas_call(
        _prep_tc,
        grid=(NBLK,),
        in_specs=[
            pl.BlockSpec((NW, 128, 1), lambda i: (0, i, 0)),
            pl.BlockSpec((128, D), lambda i: (i, 0)),
        ],
        out_specs=[
            pl.BlockSpec((128, 1), lambda i: (i, 0)),
            pl.BlockSpec((128, 1), lambda i: (i, 0)),
            pl.BlockSpec((128, D), lambda i: (i, 0)),
        ],
        out_shape=[
            jax.ShapeDtypeStruct((NPAD, 1), jnp.float32),
            jax.ShapeDtypeStruct((NPAD, 1), jnp.float32),
            jax.ShapeDtypeStruct((NPAD, D), jnp.float32),
        ],
    )(hists.reshape(NW, NPAD, 1), xpad)


def _scale_rows(rows, wbuf, nrows):
    def grp(g, _):
        wv = wbuf[pl.ds(g * L, L)]
        for j in range(L):
            cj = _splat(wv, j)
            for k in range(D // L):
                sl = pl.ds(k * L, L)
                rows[g * L + j, sl] = rows[g * L + j, sl] * cj
        return 0

    lax.fori_loop(0, nrows // L, grp, 0)


def _hop_sc(g_hbm, row_hbm, col_hbm, ew_hbm, out_hbm,
            rb0, rb1, cb0, cb1, eb0, eb1, rows0, rows1,
            rbE, cbE, ebE,
            acc, sg0, sg1, ss0, ss1, si0, si1):
    c = lax.axis_index("c")
    s = lax.axis_index("s")
    wid = c * NS + s
    eper = E // NW
    r0 = s * (NPAD // NS)
    CH = 128
    NCHUNK = 78
    ebase = wid * eper

    _zero_rows(rows0, 128, D)
    _acc_zero_and_writeback(rows0, acc, None, r0, True)
    plsc.subcore_barrier()

    rb = (rb0, rb1)
    cb = (cb0, cb1)
    eb = (eb0, eb1)
    rw = (rows0, rows1)
    sg = (sg0, sg1)
    ss = (ss0, ss1)
    si = (si0, si1)

    def idx_issue(t, b):
        base = ebase + t * CH
        pltpu.async_copy(row_hbm.at[pl.ds(base, CH)], rb[b], si[b])
        pltpu.async_copy(col_hbm.at[pl.ds(base, CH)], cb[b], si[b])
        pltpu.async_copy(ew_hbm.at[pl.ds(base, CH)], eb[b], si[b])

    def idx_wait(t, b):
        base = ebase + t * CH
        pltpu.make_async_copy(row_hbm.at[pl.ds(base, CH)], rb[b], si[b]).wait()
        pltpu.make_async_copy(col_hbm.at[pl.ds(base, CH)], cb[b], si[b]).wait()
        pltpu.make_async_copy(ew_hbm.at[pl.ds(base, CH)], eb[b], si[b]).wait()

    def gather_issue(b):
        pltpu.async_copy(g_hbm.at[rb[b]], rw[b], sg[b])

    def gather_wait(b):
        pltpu.make_async_copy(g_hbm.at[rb[b]], rw[b], sg[b]).wait()

    def scatter_issue(b):
        pltpu.async_copy(rw[b], acc.at[cb[b]], ss[b], add=True)

    def scatter_wait(b):
        pltpu.make_async_copy(rw[b], acc.at[cb[b]], ss[b]).wait()

    pltpu.sync_copy(row_hbm.at[pl.ds(ebase, CH)], rb0)
    pltpu.sync_copy(col_hbm.at[pl.ds(ebase, CH)], cb0)
    pltpu.sync_copy(ew_hbm.at[pl.ds(ebase, CH)], eb0)
    gather_issue(0)

    def pair(k, _):
        for b in (0, 1):
            t = k * 2 + b
            nb = 1 - b

            @pl.when(t > 0)
            def _():
                scatter_wait(nb)

            @pl.when(t < NCHUNK - 1)
            def _():
                idx_issue(t + 1, nb)
                idx_wait(t + 1, nb)
                gather_issue(nb)

            gather_wait(b)
            _scale_rows(rw[b], eb[b], CH)
            scatter_issue(b)
        return 0

    lax.fori_loop(0, NCHUNK // 2, pair, 0)
    scatter_wait(1)

    base = ebase + NCHUNK * CH
    pltpu.sync_copy(row_hbm.at[pl.ds(base, 16)], rbE)
    pltpu.sync_copy(col_hbm.at[pl.ds(base, 16)], cbE)
    pltpu.sync_copy(ew_hbm.at[pl.ds(base, 16)], ebE)
    pltpu.async_copy(g_hbm.at[rbE], rows0.at[pl.ds(0, 16)], sg0).wait()
    _scale_rows(rows0, ebE, 16)
    pltpu.sync_copy(rows0.at[pl.ds(0, 16)], acc.at[cbE], add=True)

    plsc.subcore_barrier()
    _acc_zero_and_writeback(rows0, acc, out_hbm.at[c], r0, False)


def _hop_kernel(g, row, col, ew):
    return pl.kernel(
        _hop_sc,
        out_type=jax.ShapeDtypeStruct((NC, NPAD, D), jnp.float32),
        mesh=_mesh(),
        compiler_params=pltpu.CompilerParams(needs_layout_passes=False),
        scratch_types=[
            pltpu.VMEM((128,), jnp.int32),
            pltpu.VMEM((128,), jnp.int32),
            pltpu.VMEM((128,), jnp.int32),
            pltpu.VMEM((128,), jnp.int32),
            pltpu.VMEM((128,), jnp.float32),
            pltpu.VMEM((128,), jnp.float32),
            pltpu.VMEM((128, D), jnp.float32),
            pltpu.VMEM((128, D), jnp.float32),
            pltpu.VMEM((16,), jnp.int32),
            pltpu.VMEM((16,), jnp.int32),
            pltpu.VMEM((16,), jnp.float32),
            pltpu.VMEM_SHARED((NPAD, D), jnp.float32),
            pltpu.SemaphoreType.DMA,
            pltpu.SemaphoreType.DMA,
            pltpu.SemaphoreType.DMA,
            pltpu.SemaphoreType.DMA,
            pltpu.SemaphoreType.DMA,
            pltpu.SemaphoreType.DMA,
        ],
    )(g, row, col, ew)


def _combine_tc(p_ref, dinv_ref, dsq_ref, h_ref, hn_ref, gn_ref):
    sm = p_ref[0] + p_ref[1]
    hn = dinv_ref[...] * sm + dsq_ref[...] * h_ref[...]
    hn_ref[...] = hn
    gn_ref[...] = dinv_ref[...] * hn


def _combine_kernel(p, dinv, dsq, h):
    return pl.pallas_call(
        _combine_tc,
        grid=(NBLK,),
        in_specs=[
            pl.BlockSpec((NC, 128, D), lambda i: (0, i, 0)),
            pl.BlockSpec((128, 1), lambda i: (i, 0)),
            pl.BlockSpec((128, 1), lambda i: (i, 0)),
            pl.BlockSpec((128, D), lambda i: (i, 0)),
        ],
        out_specs=[
            pl.BlockSpec((128, D), lambda i: (i, 0)),
            pl.BlockSpec((128, D), lambda i: (i, 0)),
        ],
        out_shape=[
            jax.ShapeDtypeStruct((NPAD, D), jnp.float32),
            jax.ShapeDtypeStruct((NPAD, D), jnp.float32),
        ],
    )(p, dinv, dsq, h)


def _mm_tc(p_ref, dinv_ref, dsq_ref, h1_ref, wsg_ref, bsg_ref, wgat_ref,
           asrc_ref, adst_ref, xh_ref, a4_ref):
    h2 = dinv_ref[...] * (p_ref[0] + p_ref[1]) + dsq_ref[...] * h1_ref[...]
    t = jnp.dot(h2, wsg_ref[...],
                preferred_element_type=jnp.float32) + bsg_ref[...]
    t = _SELU_L * jnp.where(t > 0, t, _SELU_A * (jnp.exp(t) - 1.0))
    y = jnp.dot(t, wgat_ref[...], preferred_element_type=jnp.float32)
    y0 = y[:, :OUT]
    y1 = y[:, OUT:]
    xh_ref[0] = y0
    xh_ref[1] = y1
    a4_ref[...] = jnp.concatenate([
        jnp.sum(y0 * asrc_ref[0:1, :], axis=1, keepdims=True),
        jnp.sum(y1 * asrc_ref[1:2, :], axis=1, keepdims=True),
        jnp.sum(y0 * adst_ref[0:1, :], axis=1, keepdims=True),
        jnp.sum(y1 * adst_ref[1:2, :], axis=1, keepdims=True),
    ], axis=1)


def _mm_kernel(p2, dinv, dsq, h1, W_sg, b_sg, W_gat, att_src, att_dst):
    return pl.pallas_call(
        _mm_tc,
        grid=(NBLK,),
        in_specs=[
            pl.BlockSpec((NC, 128, D), lambda i: (0, i, 0)),
            pl.BlockSpec((128, 1), lambda i: (i, 0)),
            pl.BlockSpec((128, 1), lambda i: (i, 0)),
            pl.BlockSpec((128, D), lambda i: (i, 0)),
            pl.BlockSpec((D, HID), lambda i: (0, 0)),
            pl.BlockSpec((1, HID), lambda i: (0, 0)),
            pl.BlockSpec((HID, NH * OUT), lambda i: (0, 0)),
            pl.BlockSpec((NH, OUT), lambda i: (0, 0)),
            pl.BlockSpec((NH, OUT), lambda i: (0, 0)),
        ],
        out_specs=[
            pl.BlockSpec((NH, 128, OUT), lambda i: (0, i, 0)),
            pl.BlockSpec((128, 4), lambda i: (i, 0)),
        ],
        out_shape=[
            jax.ShapeDtypeStruct((NH, NPAD, OUT), jnp.float32),
            jax.ShapeDtypeStruct((NPAD, 4), jnp.float32),
        ],
    )(p2, dinv, dsq, h1, W_sg, b_sg.reshape(1, HID), W_gat, att_src, att_dst)


def _gat_sc(xh_hbm, row_hbm, col_hbm, asrc_hbm, adst_hbm,
            o_hbm, ah_hbm,
            rb0, rb1, cb0, cb1, ab0, ab1, rows0, rows1, rbE, cbE,
            asrcv, adstv, hist, acc, sg0, sg1, ss0, ss1, si0, si1):
    c = lax.axis_index("c")
    s = lax.axis_index("s")
    eper = E // NS
    cbase = c * NPAD
    r0 = s * (NPAD // NS)
    CH = 64
    NCHUNK = 312
    ebase = s * eper

    pltpu.sync_copy(asrc_hbm.at[c], asrcv)
    pltpu.sync_copy(adst_hbm.at[c], adstv)
    _zero_ref_1d(hist, NPAD)
    _zero_rows(rows0, CH, OUT)
    _zero_rows(rows1, CH, OUT)
    for k in range(9):
        pltpu.sync_copy(rows0, acc.at[pl.ds(r0 + k * CH, CH)])
    pltpu.sync_copy(rows0.at[pl.ds(0, 56)], acc.at[pl.ds(r0 + 576, 56)])
    plsc.subcore_barrier()

    rb = (rb0, rb1)
    cb = (cb0, cb1)
    ab = (ab0, ab1)
    rw = (rows0, rows1)
    sg = (sg0, sg1)
    ss = (ss0, ss1)
    si = (si0, si1)

    def idx_issue(t, b):
        base = ebase + t * CH
        pltpu.async_copy(row_hbm.at[pl.ds(base, CH)], rb[b], si[b])
        pltpu.async_copy(col_hbm.at[pl.ds(base, CH)], cb[b], si[b])

    def idx_wait(t, b):
        base = ebase + t * CH
        pltpu.make_async_copy(row_hbm.at[pl.ds(base, CH)], rb[b], si[b]).wait()
        pltpu.make_async_copy(col_hbm.at[pl.ds(base, CH)], cb[b], si[b]).wait()

    def alpha_chunk(b, nrows):
        def grp(g, _):
            sl = pl.ds(g * L, L)
            rv = rb[b][sl]
            cv = cb[b][sl]
            sv = plsc.load_gather(asrcv, [rv])
            dv = plsc.load_gather(adstv, [cv])
            t = sv + dv
            t = jnp.where(t >= 0, t, 0.2 * t)
            al = jnp.exp(t)
            ab[b][sl] = al
            plsc.addupdate_scatter(hist, [cv], al)
            rb[b][sl] = rv + cbase
            return 0

        lax.fori_loop(0, nrows // L, grp, 0)

    def gather_issue(b):
        pltpu.async_copy(xh_hbm.at[rb[b]], rw[b], sg[b])

    def gather_wait(b):
        pltpu.make_async_copy(xh_hbm.at[rb[b]], rw[b], sg[b]).wait()

    def scatter_issue(b):
        pltpu.async_copy(rw[b], acc.at[cb[b]], ss[b], add=True)

    def scatter_wait(b):
        pltpu.make_async_copy(rw[b], acc.at[cb[b]], ss[b]).wait()

    pltpu.sync_copy(row_hbm.at[pl.ds(ebase, CH)], rb0)
    pltpu.sync_copy(col_hbm.at[pl.ds(ebase, CH)], cb0)
    alpha_chunk(0, CH)
    gather_issue(0)

    def pair(k, _):
        for b in (0, 1):
            t = k * 2 + b
            nb = 1 - b

            @pl.when(t > 0)
            def _():
                scatter_wait(nb)

            @pl.when(t < NCHUNK - 1)
            def _():
                idx_issue(t + 1, nb)
                idx_wait(t + 1, nb)
                alpha_chunk(nb, CH)
                gather_issue(nb)

            gather_wait(b)
            _scale_rows(rw[b], ab[b], CH)
            scatter_issue(b)
        return 0

    lax.fori_loop(0, NCHUNK // 2, pair, 0)
    scatter_wait(1)

    base = ebase + NCHUNK * CH
    pltpu.sync_copy(row_hbm.at[pl.ds(base, 32)], rbE)
    pltpu.sync_copy(col_hbm.at[pl.ds(base, 32)], cbE)

    def tgrp(g, _):
        sl = pl.ds(g * L, L)
        rv = rbE[sl]
        cv = cbE[sl]
        sv = plsc.load_gather(asrcv, [rv])
        dv = plsc.load_gather(adstv, [cv])
        t = sv + dv
        t = jnp.where(t >= 0, t, 0.2 * t)
        al = jnp.exp(t)
        ab0[sl] = al
        plsc.addupdate_scatter(hist, [cv], al)
        rbE[sl] = rv + cbase
        return 0

    lax.fori_loop(0, 2, tgrp, 0)
    pltpu.async_copy(xh_hbm.at[rbE], rows0.at[pl.ds(0, 32)], sg0).wait()
    _scale_rows(rows0, ab0, 32)
    pltpu.sync_copy(rows0.at[pl.ds(0, 32)], acc.at[cbE], add=True)

    pltpu.sync_copy(hist, ah_hbm.at[c, s])
    plsc.subcore_barrier()
    for k in range(4):
        pltpu.sync_copy(acc.at[pl.ds(r0 + k * 128, 128)],
                        o_hbm.at[c, pl.ds(r0 + k * 128, 128)])
    pltpu.sync_copy(acc.at[pl.ds(r0 + 512, 120)],
                    o_hbm.at[c, pl.ds(r0 + 512, 120)])


def _gat_kernel(xhflat, row, col, asrc2, adst2):
    return pl.kernel(
        _gat_sc,
        out_type=[
            jax.ShapeDtypeStruct((NC, NPAD, OUT), jnp.float32),
            jax.ShapeDtypeStruct((NC, NS, NPAD), jnp.float32),
        ],
        mesh=_mesh(),
        compiler_params=pltpu.CompilerParams(needs_layout_passes=False),
        scratch_types=[
            pltpu.VMEM((64,), jnp.int32),
            pltpu.VMEM((64,), jnp.int32),
            pltpu.VMEM((64,), jnp.int32),
            pltpu.VMEM((64,), jnp.int32),
            pltpu.VMEM((64,), jnp.float32),
            pltpu.VMEM((64,), jnp.float32),
            pltpu.VMEM((64, OUT), jnp.float32),
            pltpu.VMEM((64, OUT), jnp.float32),
            pltpu.VMEM((32,), jnp.int32),
            pltpu.VMEM((32,), jnp.int32),
            pltpu.VMEM((NPAD,), jnp.float32),
            pltpu.VMEM((NPAD,), jnp.float32),
            pltpu.VMEM((NPAD,), jnp.float32),
            pltpu.VMEM_SHARED((NPAD, OUT), jnp.float32),
            pltpu.SemaphoreType.DMA,
            pltpu.SemaphoreType.DMA,
            pltpu.SemaphoreType.DMA,
            pltpu.SemaphoreType.DMA,
            pltpu.SemaphoreType.DMA,
            pltpu.SemaphoreType.DMA,
        ],
    )(xhflat, row, col, asrc2, adst2)


def _final_tc(o_ref, ah_ref, a4_ref, xh_ref, bg_ref, out_ref):
    a4 = a4_ref[...]
    t0 = a4[:, 0:1] + a4[:, 2:3]
    t1 = a4[:, 1:2] + a4[:, 3:4]
    aL0 = jnp.exp(jnp.where(t0 >= 0, t0, 0.2 * t0))
    aL1 = jnp.exp(jnp.where(t1 >= 0, t1, 0.2 * t1))
    den0 = jnp.sum(ah_ref[0], axis=0) + aL0 + 1e-16
    den1 = jnp.sum(ah_ref[1], axis=0) + aL1 + 1e-16
    num0 = o_ref[0] + aL0 * xh_ref[0]
    num1 = o_ref[1] + aL1 * xh_ref[1]
    out_ref[...] = 0.5 * (num0 / den0 + num1 / den1) + bg_ref[...]


def _final_kernel(o, ah, a4, xh, b_gat):
    return pl.pallas_call(
        _final_tc,
        grid=(NBLK,),
        in_specs=[
            pl.BlockSpec((NC, 128, OUT), lambda i: (0, i, 0)),
            pl.BlockSpec((NC, NS, 128, 1), lambda i: (0, 0, i, 0)),
            pl.BlockSpec((128, 4), lambda i: (i, 0)),
            pl.BlockSpec((NC, 128, OUT), lambda i: (0, i, 0)),
            pl.BlockSpec((1, OUT), lambda i: (0, 0)),
        ],
        out_specs=pl.BlockSpec((128, OUT), lambda i: (i, 0)),
        out_shape=jax.ShapeDtypeStruct((NPAD, OUT), jnp.float32),
    )(o, ah.reshape(NC, NS, NPAD, 1), a4, xh, b_gat.reshape(1, OUT))


def kernel(x, edge_index, edge_attr, W_sg, b_sg, W_gat, att_src, att_dst,
           b_gat):
    row = edge_index[0]
    col = edge_index[1]
    xpad = jnp.pad(x, ((0, NPAD - N), (0, 0)))

    hists = _deg_kernel(col, edge_attr)
    dinv, dsq, g = _prep_kernel(hists, xpad)
    p1 = _hop_kernel(g, row, col, edge_attr)
    h1, g1 = _combine_kernel(p1, dinv, dsq, xpad)
    p2 = _hop_kernel(g1, row, col, edge_attr)

    xh, a4 = _mm_kernel(p2, dinv, dsq, h1, W_sg, b_sg, W_gat, att_src,
                        att_dst)
    asrc2 = jnp.transpose(a4[:, 0:2])
    adst2 = jnp.transpose(a4[:, 2:4])
    o, ah = _gat_kernel(xh.reshape(NC * NPAD, OUT), row, col, asrc2, adst2)
    out = _final_kernel(o, ah, a4, xh, b_gat)
    return out[:N]


# speedup vs baseline: 41.5255x; 1.0113x over previous
import jax
import jax.numpy as jnp
from jax import lax
from jax.experimental import pallas as pl
from jax.experimental.pallas import tpu as pltpu
from jax.experimental.pallas import tpu_sc as plsc

N = 10000
E = 320000
D = 128
HID = 64
OUT = 128
NH = 2

NC, NS, L = 2, 16, 16
NW = NC * NS
NPAD = 10112
NBLK = NPAD // 128

_SELU_L = 1.0507009873554805
_SELU_A = 1.6732632423543772


def _mesh():
    return plsc.VectorSubcoreMesh(core_axis_name="c", subcore_axis_name="s",
                                  num_cores=NC, num_subcores=NS)


def _splat(v, j):
    return lax.gather(
        v, jnp.full((L, 1), j, jnp.int32),
        dimension_numbers=lax.GatherDimensionNumbers(
            offset_dims=(), collapsed_slice_dims=(0,), start_index_map=(0,)),
        slice_sizes=(1,), mode=lax.GatherScatterMode.PROMISE_IN_BOUNDS)


def _zero_ref_1d(ref, n):
    z = jnp.zeros((L,), jnp.float32)

    def body(i, _):
        ref[pl.ds(i * L, L)] = z
        return 0

    lax.fori_loop(0, n // L, body, 0)


def _zero_rows(ref, nrows, width):
    z = jnp.zeros((L,), jnp.float32)

    def body(i, _):
        for k in range(width // L):
            ref[i, pl.ds(k * L, L)] = z
        return 0

    lax.fori_loop(0, nrows, body, 0)


def _acc_zero_and_writeback(rows, acc, dst, r0, zero_phase):
    if zero_phase:
        for k in range(4):
            pltpu.sync_copy(rows.at[pl.ds(0, 128)],
                            acc.at[pl.ds(r0 + k * 128, 128)])
        pltpu.sync_copy(rows.at[pl.ds(0, 120)], acc.at[pl.ds(r0 + 512, 120)])
    else:
        for k in range(4):
            pltpu.sync_copy(acc.at[pl.ds(r0 + k * 128, 128)],
                            dst.at[pl.ds(r0 + k * 128, 128)])
        pltpu.sync_copy(acc.at[pl.ds(r0 + 512, 120)],
                        dst.at[pl.ds(r0 + 512, 120)])


def _deg_sc(col_hbm, ew_hbm, out_hbm, colbuf, ewbuf, hist):
    c = lax.axis_index("c")
    s = lax.axis_index("s")
    wid = c * NS + s
    _zero_ref_1d(hist, NPAD)
    eper = E // NW
    chunk = 2000

    def chunk_body(t, _):
        base = wid * eper + t * chunk
        pltpu.sync_copy(col_hbm.at[pl.ds(base, chunk)], colbuf)
        pltpu.sync_copy(ew_hbm.at[pl.ds(base, chunk)], ewbuf)

        def grp(g, _):
            cv = colbuf[pl.ds(g * L, L)]
            ev = ewbuf[pl.ds(g * L, L)]
            plsc.addupdate_scatter(hist, [cv], ev)
            return 0

        lax.fori_loop(0, chunk // L, grp, 0)
        return 0

    lax.fori_loop(0, eper // chunk, chunk_body, 0)
    pltpu.sync_copy(hist, out_hbm.at[wid])


def _deg_kernel(col, ew):
    return pl.kernel(
        _deg_sc,
        out_type=jax.ShapeDtypeStruct((NW, NPAD), jnp.float32),
        mesh=_mesh(),
        compiler_params=pltpu.CompilerParams(needs_layout_passes=False),
        scratch_types=[
            pltpu.VMEM((2000,), jnp.int32),
            pltpu.VMEM((2000,), jnp.float32),
            pltpu.VMEM((NPAD,), jnp.float32),
        ],
    )(col, ew)


def _prep_tc(hist_ref, x_ref, dinv_ref, dsq_ref, g_ref):
    deg = jnp.sum(hist_ref[...], axis=0) + 1.0
    dinv = lax.rsqrt(deg)
    dinv_ref[...] = dinv
    dsq_ref[...] = 1.0 / deg
    g_ref[...] = x_ref[...] * dinv


def _prep_kernel(hists, xpad):
    return pl.pallas_call(
        _prep_tc,
        grid=(NBLK,),
        in_specs=[
            pl.BlockSpec((NW, 128, 1), lambda i: (0, i, 0)),
            pl.BlockSpec((128, D), lambda i: (i, 0)),
        ],
        out_specs=[
            pl.BlockSpec((128, 1), lambda i: (i, 0)),
            pl.BlockSpec((128, 1), lambda i: (i, 0)),
            pl.BlockSpec((128, D), lambda i: (i, 0)),
        ],
        out_shape=[
            jax.ShapeDtypeStruct((NPAD, 1), jnp.float32),
            jax.ShapeDtypeStruct((NPAD, 1), jnp.float32),
            jax.ShapeDtypeStruct((NPAD, D), jnp.float32),
        ],
    )(hists.reshape(NW, NPAD, 1), xpad)


def _scale_rows(rows, wbuf, nrows):
    def grp(g, _):
        wv = wbuf[pl.ds(g * L, L)]
        for j in range(L):
            cj = _splat(wv, j)
            for k in range(D // L):
                sl = pl.ds(k * L, L)
                rows[g * L + j, sl] = rows[g * L + j, sl] * cj
        return 0

    lax.fori_loop(0, nrows // L, grp, 0)


def _hop_sc(g_hbm, row_hbm, col_hbm, ew_hbm, out_hbm,
            rb0, rb1, cb0, cb1, eb0, eb1, rows0, rows1,
            rbE, cbE, ebE,
            acc, sg0, sg1, ss0, ss1, si0, si1):
    c = lax.axis_index("c")
    s = lax.axis_index("s")
    wid = c * NS + s
    eper = E // NW
    r0 = s * (NPAD // NS)
    CH = 160
    NCHUNK = 62
    ebase = wid * eper

    _zero_rows(rows0, 128, D)
    _acc_zero_and_writeback(rows0, acc, None, r0, True)
    plsc.subcore_barrier()

    rb = (rb0, rb1)
    cb = (cb0, cb1)
    eb = (eb0, eb1)
    rw = (rows0, rows1)
    sg = (sg0, sg1)
    ss = (ss0, ss1)
    si = (si0, si1)

    def idx_issue(t, b):
        base = ebase + t * CH
        pltpu.async_copy(row_hbm.at[pl.ds(base, CH)], rb[b], si[b])
        pltpu.async_copy(col_hbm.at[pl.ds(base, CH)], cb[b], si[b])
        pltpu.async_copy(ew_hbm.at[pl.ds(base, CH)], eb[b], si[b])

    def idx_wait(t, b):
        base = ebase + t * CH
        pltpu.make_async_copy(row_hbm.at[pl.ds(base, CH)], rb[b], si[b]).wait()
        pltpu.make_async_copy(col_hbm.at[pl.ds(base, CH)], cb[b], si[b]).wait()
        pltpu.make_async_copy(ew_hbm.at[pl.ds(base, CH)], eb[b], si[b]).wait()

    def gather_issue(b):
        pltpu.async_copy(g_hbm.at[rb[b]], rw[b], sg[b])

    def gather_wait(b):
        pltpu.make_async_copy(g_hbm.at[rb[b]], rw[b], sg[b]).wait()

    def scatter_issue(b):
        pltpu.async_copy(rw[b], acc.at[cb[b]], ss[b], add=True)

    def scatter_wait(b):
        pltpu.make_async_copy(rw[b], acc.at[cb[b]], ss[b]).wait()

    pltpu.sync_copy(row_hbm.at[pl.ds(ebase, CH)], rb0)
    pltpu.sync_copy(col_hbm.at[pl.ds(ebase, CH)], cb0)
    pltpu.sync_copy(ew_hbm.at[pl.ds(ebase, CH)], eb0)
    gather_issue(0)

    def pair(k, _):
        for b in (0, 1):
            t = k * 2 + b
            nb = 1 - b

            @pl.when(t > 0)
            def _():
                scatter_wait(nb)

            @pl.when(t < NCHUNK - 1)
            def _():
                idx_issue(t + 1, nb)
                idx_wait(t + 1, nb)
                gather_issue(nb)

            gather_wait(b)
            _scale_rows(rw[b], eb[b], CH)
            scatter_issue(b)
        return 0

    lax.fori_loop(0, NCHUNK // 2, pair, 0)
    scatter_wait(1)

    base = ebase + NCHUNK * CH
    pltpu.sync_copy(row_hbm.at[pl.ds(base, 80)], rbE)
    pltpu.sync_copy(col_hbm.at[pl.ds(base, 80)], cbE)
    pltpu.sync_copy(ew_hbm.at[pl.ds(base, 80)], ebE)
    pltpu.async_copy(g_hbm.at[rbE], rows0.at[pl.ds(0, 80)], sg0).wait()
    _scale_rows(rows0, ebE, 80)
    pltpu.sync_copy(rows0.at[pl.ds(0, 80)], acc.at[cbE], add=True)

    plsc.subcore_barrier()
    _acc_zero_and_writeback(rows0, acc, out_hbm.at[c], r0, False)


def _hop_kernel(g, row, col, ew):
    return pl.kernel(
        _hop_sc,
        out_type=jax.ShapeDtypeStruct((NC, NPAD, D), jnp.float32),
        mesh=_mesh(),
        compiler_params=pltpu.CompilerParams(needs_layout_passes=False),
        scratch_types=[
            pltpu.VMEM((160,), jnp.int32),
            pltpu.VMEM((160,), jnp.int32),
            pltpu.VMEM((160,), jnp.int32),
            pltpu.VMEM((160,), jnp.int32),
            pltpu.VMEM((160,), jnp.float32),
            pltpu.VMEM((160,), jnp.float32),
            pltpu.VMEM((160, D), jnp.float32),
            pltpu.VMEM((160, D), jnp.float32),
            pltpu.VMEM((80,), jnp.int32),
            pltpu.VMEM((80,), jnp.int32),
            pltpu.VMEM((80,), jnp.float32),
            pltpu.VMEM_SHARED((NPAD, D), jnp.float32),
            pltpu.SemaphoreType.DMA,
            pltpu.SemaphoreType.DMA,
            pltpu.SemaphoreType.DMA,
            pltpu.SemaphoreType.DMA,
            pltpu.SemaphoreType.DMA,
            pltpu.SemaphoreType.DMA,
        ],
    )(g, row, col, ew)


def _combine_tc(p_ref, dinv_ref, dsq_ref, h_ref, hn_ref, gn_ref):
    sm = p_ref[0] + p_ref[1]
    hn = dinv_ref[...] * sm + dsq_ref[...] * h_ref[...]
    hn_ref[...] = hn
    gn_ref[...] = dinv_ref[...] * hn


def _combine_kernel(p, dinv, dsq, h):
    return pl.pallas_call(
        _combine_tc,
        grid=(NBLK,),
        in_specs=[
            pl.BlockSpec((NC, 128, D), lambda i: (0, i, 0)),
            pl.BlockSpec((128, 1), lambda i: (i, 0)),
            pl.BlockSpec((128, 1), lambda i: (i, 0)),
            pl.BlockSpec((128, D), lambda i: (i, 0)),
        ],
        out_specs=[
            pl.BlockSpec((128, D), lambda i: (i, 0)),
            pl.BlockSpec((128, D), lambda i: (i, 0)),
        ],
        out_shape=[
            jax.ShapeDtypeStruct((NPAD, D), jnp.float32),
            jax.ShapeDtypeStruct((NPAD, D), jnp.float32),
        ],
    )(p, dinv, dsq, h)


def _mm_tc(p_ref, dinv_ref, dsq_ref, h1_ref, wsg_ref, bsg_ref, wgat_ref,
           asrc_ref, adst_ref, xh_ref, a4_ref):
    h2 = dinv_ref[...] * (p_ref[0] + p_ref[1]) + dsq_ref[...] * h1_ref[...]
    t = jnp.dot(h2, wsg_ref[...],
                preferred_element_type=jnp.float32) + bsg_ref[...]
    t = _SELU_L * jnp.where(t > 0, t, _SELU_A * (jnp.exp(t) - 1.0))
    y = jnp.dot(t, wgat_ref[...], preferred_element_type=jnp.float32)
    y0 = y[:, :OUT]
    y1 = y[:, OUT:]
    xh_ref[0] = y0
    xh_ref[1] = y1
    a4_ref[...] = jnp.concatenate([
        jnp.sum(y0 * asrc_ref[0:1, :], axis=1, keepdims=True),
        jnp.sum(y1 * asrc_ref[1:2, :], axis=1, keepdims=True),
        jnp.sum(y0 * adst_ref[0:1, :], axis=1, keepdims=True),
        jnp.sum(y1 * adst_ref[1:2, :], axis=1, keepdims=True),
    ], axis=1)


def _mm_kernel(p2, dinv, dsq, h1, W_sg, b_sg, W_gat, att_src, att_dst):
    return pl.pallas_call(
        _mm_tc,
        grid=(NBLK,),
        in_specs=[
            pl.BlockSpec((NC, 128, D), lambda i: (0, i, 0)),
            pl.BlockSpec((128, 1), lambda i: (i, 0)),
            pl.BlockSpec((128, 1), lambda i: (i, 0)),
            pl.BlockSpec((128, D), lambda i: (i, 0)),
            pl.BlockSpec((D, HID), lambda i: (0, 0)),
            pl.BlockSpec((1, HID), lambda i: (0, 0)),
            pl.BlockSpec((HID, NH * OUT), lambda i: (0, 0)),
            pl.BlockSpec((NH, OUT), lambda i: (0, 0)),
            pl.BlockSpec((NH, OUT), lambda i: (0, 0)),
        ],
        out_specs=[
            pl.BlockSpec((NH, 128, OUT), lambda i: (0, i, 0)),
            pl.BlockSpec((128, 4), lambda i: (i, 0)),
        ],
        out_shape=[
            jax.ShapeDtypeStruct((NH, NPAD, OUT), jnp.float32),
            jax.ShapeDtypeStruct((NPAD, 4), jnp.float32),
        ],
    )(p2, dinv, dsq, h1, W_sg, b_sg.reshape(1, HID), W_gat, att_src, att_dst)


def _gat_sc(xh_hbm, row_hbm, col_hbm, asrc_hbm, adst_hbm,
            o_hbm, ah_hbm,
            rb0, rb1, cb0, cb1, ab0, ab1, rows0, rows1, rbE, cbE,
            asrcv, adstv, hist, acc, sg0, sg1, ss0, ss1, si0, si1):
    c = lax.axis_index("c")
    s = lax.axis_index("s")
    eper = E // NS
    cbase = c * NPAD
    r0 = s * (NPAD // NS)
    CH = 64
    NCHUNK = 312
    ebase = s * eper

    pltpu.sync_copy(asrc_hbm.at[c], asrcv)
    pltpu.sync_copy(adst_hbm.at[c], adstv)
    _zero_ref_1d(hist, NPAD)
    _zero_rows(rows0, CH, OUT)
    _zero_rows(rows1, CH, OUT)
    for k in range(9):
        pltpu.sync_copy(rows0, acc.at[pl.ds(r0 + k * CH, CH)])
    pltpu.sync_copy(rows0.at[pl.ds(0, 56)], acc.at[pl.ds(r0 + 576, 56)])
    plsc.subcore_barrier()

    rb = (rb0, rb1)
    cb = (cb0, cb1)
    ab = (ab0, ab1)
    rw = (rows0, rows1)
    sg = (sg0, sg1)
    ss = (ss0, ss1)
    si = (si0, si1)

    def idx_issue(t, b):
        base = ebase + t * CH
        pltpu.async_copy(row_hbm.at[pl.ds(base, CH)], rb[b], si[b])
        pltpu.async_copy(col_hbm.at[pl.ds(base, CH)], cb[b], si[b])

    def idx_wait(t, b):
        base = ebase + t * CH
        pltpu.make_async_copy(row_hbm.at[pl.ds(base, CH)], rb[b], si[b]).wait()
        pltpu.make_async_copy(col_hbm.at[pl.ds(base, CH)], cb[b], si[b]).wait()

    def alpha_chunk(b, nrows):
        def grp(g, _):
            sl = pl.ds(g * L, L)
            rv = rb[b][sl]
            cv = cb[b][sl]
            sv = plsc.load_gather(asrcv, [rv])
            dv = plsc.load_gather(adstv, [cv])
            t = sv + dv
            t = jnp.where(t >= 0, t, 0.2 * t)
            al = jnp.exp(t)
            ab[b][sl] = al
            plsc.addupdate_scatter(hist, [cv], al)
            rb[b][sl] = rv + cbase
            return 0

        lax.fori_loop(0, nrows // L, grp, 0)

    def gather_issue(b):
        pltpu.async_copy(xh_hbm.at[rb[b]], rw[b], sg[b])

    def gather_wait(b):
        pltpu.make_async_copy(xh_hbm.at[rb[b]], rw[b], sg[b]).wait()

    def scatter_issue(b):
        pltpu.async_copy(rw[b], acc.at[cb[b]], ss[b], add=True)

    def scatter_wait(b):
        pltpu.make_async_copy(rw[b], acc.at[cb[b]], ss[b]).wait()

    pltpu.sync_copy(row_hbm.at[pl.ds(ebase, CH)], rb0)
    pltpu.sync_copy(col_hbm.at[pl.ds(ebase, CH)], cb0)
    alpha_chunk(0, CH)
    gather_issue(0)

    def pair(k, _):
        for b in (0, 1):
            t = k * 2 + b
            nb = 1 - b

            @pl.when(t > 0)
            def _():
                scatter_wait(nb)

            @pl.when(t < NCHUNK - 1)
            def _():
                idx_issue(t + 1, nb)
                idx_wait(t + 1, nb)
                alpha_chunk(nb, CH)
                gather_issue(nb)

            gather_wait(b)
            _scale_rows(rw[b], ab[b], CH)
            scatter_issue(b)
        return 0

    lax.fori_loop(0, NCHUNK // 2, pair, 0)
    scatter_wait(1)

    base = ebase + NCHUNK * CH
    pltpu.sync_copy(row_hbm.at[pl.ds(base, 32)], rbE)
    pltpu.sync_copy(col_hbm.at[pl.ds(base, 32)], cbE)

    def tgrp(g, _):
        sl = pl.ds(g * L, L)
        rv = rbE[sl]
        cv = cbE[sl]
        sv = plsc.load_gather(asrcv, [rv])
        dv = plsc.load_gather(adstv, [cv])
        t = sv + dv
        t = jnp.where(t >= 0, t, 0.2 * t)
        al = jnp.exp(t)
        ab0[sl] = al
        plsc.addupdate_scatter(hist, [cv], al)
        rbE[sl] = rv + cbase
        return 0

    lax.fori_loop(0, 2, tgrp, 0)
    pltpu.async_copy(xh_hbm.at[rbE], rows0.at[pl.ds(0, 32)], sg0).wait()
    _scale_rows(rows0, ab0, 32)
    pltpu.sync_copy(rows0.at[pl.ds(0, 32)], acc.at[cbE], add=True)

    pltpu.sync_copy(hist, ah_hbm.at[c, s])
    plsc.subcore_barrier()
    for k in range(4):
        pltpu.sync_copy(acc.at[pl.ds(r0 + k * 128, 128)],
                        o_hbm.at[c, pl.ds(r0 + k * 128, 128)])
    pltpu.sync_copy(acc.at[pl.ds(r0 + 512, 120)],
                    o_hbm.at[c, pl.ds(r0 + 512, 120)])


def _gat_kernel(xhflat, row, col, asrc2, adst2):
    return pl.kernel(
        _gat_sc,
        out_type=[
            jax.ShapeDtypeStruct((NC, NPAD, OUT), jnp.float32),
            jax.ShapeDtypeStruct((NC, NS, NPAD), jnp.float32),
        ],
        mesh=_mesh(),
        compiler_params=pltpu.CompilerParams(needs_layout_passes=False),
        scratch_types=[
            pltpu.VMEM((64,), jnp.int32),
            pltpu.VMEM((64,), jnp.int32),
            pltpu.VMEM((64,), jnp.int32),
            pltpu.VMEM((64,), jnp.int32),
            pltpu.VMEM((64,), jnp.float32),
            pltpu.VMEM((64,), jnp.float32),
            pltpu.VMEM((64, OUT), jnp.float32),
            pltpu.VMEM((64, OUT), jnp.float32),
            pltpu.VMEM((32,), jnp.int32),
            pltpu.VMEM((32,), jnp.int32),
            pltpu.VMEM((NPAD,), jnp.float32),
            pltpu.VMEM((NPAD,), jnp.float32),
            pltpu.VMEM((NPAD,), jnp.float32),
            pltpu.VMEM_SHARED((NPAD, OUT), jnp.float32),
            pltpu.SemaphoreType.DMA,
            pltpu.SemaphoreType.DMA,
            pltpu.SemaphoreType.DMA,
            pltpu.SemaphoreType.DMA,
            pltpu.SemaphoreType.DMA,
            pltpu.SemaphoreType.DMA,
        ],
    )(xhflat, row, col, asrc2, adst2)


def _final_tc(o_ref, ah_ref, a4_ref, xh_ref, bg_ref, out_ref):
    a4 = a4_ref[...]
    t0 = a4[:, 0:1] + a4[:, 2:3]
    t1 = a4[:, 1:2] + a4[:, 3:4]
    aL0 = jnp.exp(jnp.where(t0 >= 0, t0, 0.2 * t0))
    aL1 = jnp.exp(jnp.where(t1 >= 0, t1, 0.2 * t1))
    den0 = jnp.sum(ah_ref[0], axis=0) + aL0 + 1e-16
    den1 = jnp.sum(ah_ref[1], axis=0) + aL1 + 1e-16
    num0 = o_ref[0] + aL0 * xh_ref[0]
    num1 = o_ref[1] + aL1 * xh_ref[1]
    out_ref[...] = 0.5 * (num0 / den0 + num1 / den1) + bg_ref[...]


def _final_kernel(o, ah, a4, xh, b_gat):
    return pl.pallas_call(
        _final_tc,
        grid=(NBLK,),
        in_specs=[
            pl.BlockSpec((NC, 128, OUT), lambda i: (0, i, 0)),
            pl.BlockSpec((NC, NS, 128, 1), lambda i: (0, 0, i, 0)),
            pl.BlockSpec((128, 4), lambda i: (i, 0)),
            pl.BlockSpec((NC, 128, OUT), lambda i: (0, i, 0)),
            pl.BlockSpec((1, OUT), lambda i: (0, 0)),
        ],
        out_specs=pl.BlockSpec((128, OUT), lambda i: (i, 0)),
        out_shape=jax.ShapeDtypeStruct((NPAD, OUT), jnp.float32),
    )(o, ah.reshape(NC, NS, NPAD, 1), a4, xh, b_gat.reshape(1, OUT))


def kernel(x, edge_index, edge_attr, W_sg, b_sg, W_gat, att_src, att_dst,
           b_gat):
    row = edge_index[0]
    col = edge_index[1]
    xpad = jnp.pad(x, ((0, NPAD - N), (0, 0)))

    hists = _deg_kernel(col, edge_attr)
    dinv, dsq, g = _prep_kernel(hists, xpad)
    p1 = _hop_kernel(g, row, col, edge_attr)
    h1, g1 = _combine_kernel(p1, dinv, dsq, xpad)
    p2 = _hop_kernel(g1, row, col, edge_attr)

    xh, a4 = _mm_kernel(p2, dinv, dsq, h1, W_sg, b_sg, W_gat, att_src,
                        att_dst)
    asrc2 = jnp.transpose(a4[:, 0:2])
    adst2 = jnp.transpose(a4[:, 2:4])
    o, ah = _gat_kernel(xh.reshape(NC * NPAD, OUT), row, col, asrc2, adst2)
    out = _final_kernel(o, ah, a4, xh, b_gat)
    return out[:N]


# speedup vs baseline: 41.7945x; 1.0065x over previous
import jax
import jax.numpy as jnp
from jax import lax
from jax.experimental import pallas as pl
from jax.experimental.pallas import tpu as pltpu
from jax.experimental.pallas import tpu_sc as plsc

N = 10000
E = 320000
D = 128
HID = 64
OUT = 128
NH = 2

NC, NS, L = 2, 16, 16
NW = NC * NS
NPAD = 10112
NBLK = NPAD // 128

_SELU_L = 1.0507009873554805
_SELU_A = 1.6732632423543772


def _mesh():
    return plsc.VectorSubcoreMesh(core_axis_name="c", subcore_axis_name="s",
                                  num_cores=NC, num_subcores=NS)


def _splat(v, j):
    return lax.gather(
        v, jnp.full((L, 1), j, jnp.int32),
        dimension_numbers=lax.GatherDimensionNumbers(
            offset_dims=(), collapsed_slice_dims=(0,), start_index_map=(0,)),
        slice_sizes=(1,), mode=lax.GatherScatterMode.PROMISE_IN_BOUNDS)


def _zero_ref_1d(ref, n):
    z = jnp.zeros((L,), jnp.float32)

    def body(i, _):
        ref[pl.ds(i * L, L)] = z
        return 0

    lax.fori_loop(0, n // L, body, 0)


def _zero_rows(ref, nrows, width):
    z = jnp.zeros((L,), jnp.float32)

    def body(i, _):
        for k in range(width // L):
            ref[i, pl.ds(k * L, L)] = z
        return 0

    lax.fori_loop(0, nrows, body, 0)


def _acc_zero_and_writeback(rows, acc, dst, r0, zero_phase):
    if zero_phase:
        for k in range(4):
            pltpu.sync_copy(rows.at[pl.ds(0, 128)],
                            acc.at[pl.ds(r0 + k * 128, 128)])
        pltpu.sync_copy(rows.at[pl.ds(0, 120)], acc.at[pl.ds(r0 + 512, 120)])
    else:
        for k in range(4):
            pltpu.sync_copy(acc.at[pl.ds(r0 + k * 128, 128)],
                            dst.at[pl.ds(r0 + k * 128, 128)])
        pltpu.sync_copy(acc.at[pl.ds(r0 + 512, 120)],
                        dst.at[pl.ds(r0 + 512, 120)])


def _deg_sc(col_hbm, ew_hbm, out_hbm, colbuf, ewbuf, hist):
    c = lax.axis_index("c")
    s = lax.axis_index("s")
    wid = c * NS + s
    _zero_ref_1d(hist, NPAD)
    eper = E // NW
    chunk = 2000

    def chunk_body(t, _):
        base = wid * eper + t * chunk
        pltpu.sync_copy(col_hbm.at[pl.ds(base, chunk)], colbuf)
        pltpu.sync_copy(ew_hbm.at[pl.ds(base, chunk)], ewbuf)

        def grp(g, _):
            cv = colbuf[pl.ds(g * L, L)]
            ev = ewbuf[pl.ds(g * L, L)]
            plsc.addupdate_scatter(hist, [cv], ev)
            return 0

        lax.fori_loop(0, chunk // L, grp, 0)
        return 0

    lax.fori_loop(0, eper // chunk, chunk_body, 0)
    pltpu.sync_copy(hist, out_hbm.at[wid])


def _deg_kernel(col, ew):
    return pl.kernel(
        _deg_sc,
        out_type=jax.ShapeDtypeStruct((NW, NPAD), jnp.float32),
        mesh=_mesh(),
        compiler_params=pltpu.CompilerParams(needs_layout_passes=False),
        scratch_types=[
            pltpu.VMEM((2000,), jnp.int32),
            pltpu.VMEM((2000,), jnp.float32),
            pltpu.VMEM((NPAD,), jnp.float32),
        ],
    )(col, ew)


def _prep_tc(hist_ref, x_ref, dinv_ref, dsq_ref, g_ref):
    deg = jnp.sum(hist_ref[...], axis=0) + 1.0
    dinv = lax.rsqrt(deg)
    dinv_ref[...] = dinv
    dsq_ref[...] = 1.0 / deg
    g_ref[...] = x_ref[...] * dinv


def _prep_kernel(hists, xpad):
    return pl.pallas_call(
        _prep_tc,
        grid=(NBLK,),
        in_specs=[
            pl.BlockSpec((NW, 128, 1), lambda i: (0, i, 0)),
            pl.BlockSpec((128, D), lambda i: (i, 0)),
        ],
        out_specs=[
            pl.BlockSpec((128, 1), lambda i: (i, 0)),
            pl.BlockSpec((128, 1), lambda i: (i, 0)),
            pl.BlockSpec((128, D), lambda i: (i, 0)),
        ],
        out_shape=[
            jax.ShapeDtypeStruct((NPAD, 1), jnp.float32),
            jax.ShapeDtypeStruct((NPAD, 1), jnp.float32),
            jax.ShapeDtypeStruct((NPAD, D), jnp.float32),
        ],
    )(hists.reshape(NW, NPAD, 1), xpad)


def _scale_rows(rows, wbuf, nrows):
    @plsc.parallel_loop(0, nrows // L)
    def _(g):
        wv = wbuf[pl.ds(g * L, L)]
        for j in range(L):
            cj = _splat(wv, j)
            for k in range(D // L):
                sl = pl.ds(k * L, L)
                rows[g * L + j, sl] = rows[g * L + j, sl] * cj


def _hop_sc(g_hbm, row_hbm, col_hbm, ew_hbm, out_hbm,
            rb0, rb1, cb0, cb1, eb0, eb1, rows0, rows1,
            rbE, cbE, ebE,
            acc, sg0, sg1, ss0, ss1, si0, si1):
    c = lax.axis_index("c")
    s = lax.axis_index("s")
    wid = c * NS + s
    eper = E // NW
    r0 = s * (NPAD // NS)
    CH = 160
    NCHUNK = 62
    ebase = wid * eper

    _zero_rows(rows0, 128, D)
    _acc_zero_and_writeback(rows0, acc, None, r0, True)
    plsc.subcore_barrier()

    rb = (rb0, rb1)
    cb = (cb0, cb1)
    eb = (eb0, eb1)
    rw = (rows0, rows1)
    sg = (sg0, sg1)
    ss = (ss0, ss1)
    si = (si0, si1)

    def idx_issue(t, b):
        base = ebase + t * CH
        pltpu.async_copy(row_hbm.at[pl.ds(base, CH)], rb[b], si[b])
        pltpu.async_copy(col_hbm.at[pl.ds(base, CH)], cb[b], si[b])
        pltpu.async_copy(ew_hbm.at[pl.ds(base, CH)], eb[b], si[b])

    def idx_wait(t, b):
        base = ebase + t * CH
        pltpu.make_async_copy(row_hbm.at[pl.ds(base, CH)], rb[b], si[b]).wait()
        pltpu.make_async_copy(col_hbm.at[pl.ds(base, CH)], cb[b], si[b]).wait()
        pltpu.make_async_copy(ew_hbm.at[pl.ds(base, CH)], eb[b], si[b]).wait()

    def gather_issue(b):
        pltpu.async_copy(g_hbm.at[rb[b]], rw[b], sg[b])

    def gather_wait(b):
        pltpu.make_async_copy(g_hbm.at[rb[b]], rw[b], sg[b]).wait()

    def scatter_issue(b):
        pltpu.async_copy(rw[b], acc.at[cb[b]], ss[b], add=True)

    def scatter_wait(b):
        pltpu.make_async_copy(rw[b], acc.at[cb[b]], ss[b]).wait()

    pltpu.sync_copy(row_hbm.at[pl.ds(ebase, CH)], rb0)
    pltpu.sync_copy(col_hbm.at[pl.ds(ebase, CH)], cb0)
    pltpu.sync_copy(ew_hbm.at[pl.ds(ebase, CH)], eb0)
    gather_issue(0)

    def pair(k, _):
        for b in (0, 1):
            t = k * 2 + b
            nb = 1 - b

            @pl.when(t > 0)
            def _():
                scatter_wait(nb)

            @pl.when(t < NCHUNK - 1)
            def _():
                idx_issue(t + 1, nb)
                idx_wait(t + 1, nb)
                gather_issue(nb)

            gather_wait(b)
            _scale_rows(rw[b], eb[b], CH)
            scatter_issue(b)
        return 0

    lax.fori_loop(0, NCHUNK // 2, pair, 0)
    scatter_wait(1)

    base = ebase + NCHUNK * CH
    pltpu.sync_copy(row_hbm.at[pl.ds(base, 80)], rbE)
    pltpu.sync_copy(col_hbm.at[pl.ds(base, 80)], cbE)
    pltpu.sync_copy(ew_hbm.at[pl.ds(base, 80)], ebE)
    pltpu.async_copy(g_hbm.at[rbE], rows0.at[pl.ds(0, 80)], sg0).wait()
    _scale_rows(rows0, ebE, 80)
    pltpu.sync_copy(rows0.at[pl.ds(0, 80)], acc.at[cbE], add=True)

    plsc.subcore_barrier()
    _acc_zero_and_writeback(rows0, acc, out_hbm.at[c], r0, False)


def _hop_kernel(g, row, col, ew):
    return pl.kernel(
        _hop_sc,
        out_type=jax.ShapeDtypeStruct((NC, NPAD, D), jnp.float32),
        mesh=_mesh(),
        compiler_params=pltpu.CompilerParams(needs_layout_passes=False),
        scratch_types=[
            pltpu.VMEM((160,), jnp.int32),
            pltpu.VMEM((160,), jnp.int32),
            pltpu.VMEM((160,), jnp.int32),
            pltpu.VMEM((160,), jnp.int32),
            pltpu.VMEM((160,), jnp.float32),
            pltpu.VMEM((160,), jnp.float32),
            pltpu.VMEM((160, D), jnp.float32),
            pltpu.VMEM((160, D), jnp.float32),
            pltpu.VMEM((80,), jnp.int32),
            pltpu.VMEM((80,), jnp.int32),
            pltpu.VMEM((80,), jnp.float32),
            pltpu.VMEM_SHARED((NPAD, D), jnp.float32),
            pltpu.SemaphoreType.DMA,
            pltpu.SemaphoreType.DMA,
            pltpu.SemaphoreType.DMA,
            pltpu.SemaphoreType.DMA,
            pltpu.SemaphoreType.DMA,
            pltpu.SemaphoreType.DMA,
        ],
    )(g, row, col, ew)


def _combine_tc(p_ref, dinv_ref, dsq_ref, h_ref, hn_ref, gn_ref):
    sm = p_ref[0] + p_ref[1]
    hn = dinv_ref[...] * sm + dsq_ref[...] * h_ref[...]
    hn_ref[...] = hn
    gn_ref[...] = dinv_ref[...] * hn


def _combine_kernel(p, dinv, dsq, h):
    return pl.pallas_call(
        _combine_tc,
        grid=(NBLK,),
        in_specs=[
            pl.BlockSpec((NC, 128, D), lambda i: (0, i, 0)),
            pl.BlockSpec((128, 1), lambda i: (i, 0)),
            pl.BlockSpec((128, 1), lambda i: (i, 0)),
            pl.BlockSpec((128, D), lambda i: (i, 0)),
        ],
        out_specs=[
            pl.BlockSpec((128, D), lambda i: (i, 0)),
            pl.BlockSpec((128, D), lambda i: (i, 0)),
        ],
        out_shape=[
            jax.ShapeDtypeStruct((NPAD, D), jnp.float32),
            jax.ShapeDtypeStruct((NPAD, D), jnp.float32),
        ],
    )(p, dinv, dsq, h)


def _mm_tc(p_ref, dinv_ref, dsq_ref, h1_ref, wsg_ref, bsg_ref, wgat_ref,
           asrc_ref, adst_ref, xh_ref, a4_ref):
    h2 = dinv_ref[...] * (p_ref[0] + p_ref[1]) + dsq_ref[...] * h1_ref[...]
    t = jnp.dot(h2, wsg_ref[...],
                preferred_element_type=jnp.float32) + bsg_ref[...]
    t = _SELU_L * jnp.where(t > 0, t, _SELU_A * (jnp.exp(t) - 1.0))
    y = jnp.dot(t, wgat_ref[...], preferred_element_type=jnp.float32)
    y0 = y[:, :OUT]
    y1 = y[:, OUT:]
    xh_ref[0] = y0
    xh_ref[1] = y1
    a4_ref[...] = jnp.concatenate([
        jnp.sum(y0 * asrc_ref[0:1, :], axis=1, keepdims=True),
        jnp.sum(y1 * asrc_ref[1:2, :], axis=1, keepdims=True),
        jnp.sum(y0 * adst_ref[0:1, :], axis=1, keepdims=True),
        jnp.sum(y1 * adst_ref[1:2, :], axis=1, keepdims=True),
    ], axis=1)


def _mm_kernel(p2, dinv, dsq, h1, W_sg, b_sg, W_gat, att_src, att_dst):
    return pl.pallas_call(
        _mm_tc,
        grid=(NBLK,),
        in_specs=[
            pl.BlockSpec((NC, 128, D), lambda i: (0, i, 0)),
            pl.BlockSpec((128, 1), lambda i: (i, 0)),
            pl.BlockSpec((128, 1), lambda i: (i, 0)),
            pl.BlockSpec((128, D), lambda i: (i, 0)),
            pl.BlockSpec((D, HID), lambda i: (0, 0)),
            pl.BlockSpec((1, HID), lambda i: (0, 0)),
            pl.BlockSpec((HID, NH * OUT), lambda i: (0, 0)),
            pl.BlockSpec((NH, OUT), lambda i: (0, 0)),
            pl.BlockSpec((NH, OUT), lambda i: (0, 0)),
        ],
        out_specs=[
            pl.BlockSpec((NH, 128, OUT), lambda i: (0, i, 0)),
            pl.BlockSpec((128, 4), lambda i: (i, 0)),
        ],
        out_shape=[
            jax.ShapeDtypeStruct((NH, NPAD, OUT), jnp.float32),
            jax.ShapeDtypeStruct((NPAD, 4), jnp.float32),
        ],
    )(p2, dinv, dsq, h1, W_sg, b_sg.reshape(1, HID), W_gat, att_src, att_dst)


def _gat_sc(xh_hbm, row_hbm, col_hbm, asrc_hbm, adst_hbm,
            o_hbm, ah_hbm,
            rb0, rb1, cb0, cb1, ab0, ab1, rows0, rows1, rbE, cbE,
            asrcv, adstv, hist, acc, sg0, sg1, ss0, ss1, si0, si1):
    c = lax.axis_index("c")
    s = lax.axis_index("s")
    eper = E // NS
    cbase = c * NPAD
    r0 = s * (NPAD // NS)
    CH = 64
    NCHUNK = 312
    ebase = s * eper

    pltpu.sync_copy(asrc_hbm.at[c], asrcv)
    pltpu.sync_copy(adst_hbm.at[c], adstv)
    _zero_ref_1d(hist, NPAD)
    _zero_rows(rows0, CH, OUT)
    _zero_rows(rows1, CH, OUT)
    for k in range(9):
        pltpu.sync_copy(rows0, acc.at[pl.ds(r0 + k * CH, CH)])
    pltpu.sync_copy(rows0.at[pl.ds(0, 56)], acc.at[pl.ds(r0 + 576, 56)])
    plsc.subcore_barrier()

    rb = (rb0, rb1)
    cb = (cb0, cb1)
    ab = (ab0, ab1)
    rw = (rows0, rows1)
    sg = (sg0, sg1)
    ss = (ss0, ss1)
    si = (si0, si1)

    def idx_issue(t, b):
        base = ebase + t * CH
        pltpu.async_copy(row_hbm.at[pl.ds(base, CH)], rb[b], si[b])
        pltpu.async_copy(col_hbm.at[pl.ds(base, CH)], cb[b], si[b])

    def idx_wait(t, b):
        base = ebase + t * CH
        pltpu.make_async_copy(row_hbm.at[pl.ds(base, CH)], rb[b], si[b]).wait()
        pltpu.make_async_copy(col_hbm.at[pl.ds(base, CH)], cb[b], si[b]).wait()

    def alpha_chunk(b, nrows):
        def grp(g, _):
            sl = pl.ds(g * L, L)
            rv = rb[b][sl]
            cv = cb[b][sl]
            sv = plsc.load_gather(asrcv, [rv])
            dv = plsc.load_gather(adstv, [cv])
            t = sv + dv
            t = jnp.where(t >= 0, t, 0.2 * t)
            al = jnp.exp(t)
            ab[b][sl] = al
            plsc.addupdate_scatter(hist, [cv], al)
            rb[b][sl] = rv + cbase
            return 0

        lax.fori_loop(0, nrows // L, grp, 0)

    def gather_issue(b):
        pltpu.async_copy(xh_hbm.at[rb[b]], rw[b], sg[b])

    def gather_wait(b):
        pltpu.make_async_copy(xh_hbm.at[rb[b]], rw[b], sg[b]).wait()

    def scatter_issue(b):
        pltpu.async_copy(rw[b], acc.at[cb[b]], ss[b], add=True)

    def scatter_wait(b):
        pltpu.make_async_copy(rw[b], acc.at[cb[b]], ss[b]).wait()

    pltpu.sync_copy(row_hbm.at[pl.ds(ebase, CH)], rb0)
    pltpu.sync_copy(col_hbm.at[pl.ds(ebase, CH)], cb0)
    alpha_chunk(0, CH)
    gather_issue(0)

    def pair(k, _):
        for b in (0, 1):
            t = k * 2 + b
            nb = 1 - b

            @pl.when(t > 0)
            def _():
                scatter_wait(nb)

            @pl.when(t < NCHUNK - 1)
            def _():
                idx_issue(t + 1, nb)
                idx_wait(t + 1, nb)
                alpha_chunk(nb, CH)
                gather_issue(nb)

            gather_wait(b)
            _scale_rows(rw[b], ab[b], CH)
            scatter_issue(b)
        return 0

    lax.fori_loop(0, NCHUNK // 2, pair, 0)
    scatter_wait(1)

    base = ebase + NCHUNK * CH
    pltpu.sync_copy(row_hbm.at[pl.ds(base, 32)], rbE)
    pltpu.sync_copy(col_hbm.at[pl.ds(base, 32)], cbE)

    def tgrp(g, _):
        sl = pl.ds(g * L, L)
        rv = rbE[sl]
        cv = cbE[sl]
        sv = plsc.load_gather(asrcv, [rv])
        dv = plsc.load_gather(adstv, [cv])
        t = sv + dv
        t = jnp.where(t >= 0, t, 0.2 * t)
        al = jnp.exp(t)
        ab0[sl] = al
        plsc.addupdate_scatter(hist, [cv], al)
        rbE[sl] = rv + cbase
        return 0

    lax.fori_loop(0, 2, tgrp, 0)
    pltpu.async_copy(xh_hbm.at[rbE], rows0.at[pl.ds(0, 32)], sg0).wait()
    _scale_rows(rows0, ab0, 32)
    pltpu.sync_copy(rows0.at[pl.ds(0, 32)], acc.at[cbE], add=True)

    pltpu.sync_copy(hist, ah_hbm.at[c, s])
    plsc.subcore_barrier()
    for k in range(4):
        pltpu.sync_copy(acc.at[pl.ds(r0 + k * 128, 128)],
                        o_hbm.at[c, pl.ds(r0 + k * 128, 128)])
    pltpu.sync_copy(acc.at[pl.ds(r0 + 512, 120)],
                    o_hbm.at[c, pl.ds(r0 + 512, 120)])


def _gat_kernel(xhflat, row, col, asrc2, adst2):
    return pl.kernel(
        _gat_sc,
        out_type=[
            jax.ShapeDtypeStruct((NC, NPAD, OUT), jnp.float32),
            jax.ShapeDtypeStruct((NC, NS, NPAD), jnp.float32),
        ],
        mesh=_mesh(),
        compiler_params=pltpu.CompilerParams(needs_layout_passes=False),
        scratch_types=[
            pltpu.VMEM((64,), jnp.int32),
            pltpu.VMEM((64,), jnp.int32),
            pltpu.VMEM((64,), jnp.int32),
            pltpu.VMEM((64,), jnp.int32),
            pltpu.VMEM((64,), jnp.float32),
            pltpu.VMEM((64,), jnp.float32),
            pltpu.VMEM((64, OUT), jnp.float32),
            pltpu.VMEM((64, OUT), jnp.float32),
            pltpu.VMEM((32,), jnp.int32),
            pltpu.VMEM((32,), jnp.int32),
            pltpu.VMEM((NPAD,), jnp.float32),
            pltpu.VMEM((NPAD,), jnp.float32),
            pltpu.VMEM((NPAD,), jnp.float32),
            pltpu.VMEM_SHARED((NPAD, OUT), jnp.float32),
            pltpu.SemaphoreType.DMA,
            pltpu.SemaphoreType.DMA,
            pltpu.SemaphoreType.DMA,
            pltpu.SemaphoreType.DMA,
            pltpu.SemaphoreType.DMA,
            pltpu.SemaphoreType.DMA,
        ],
    )(xhflat, row, col, asrc2, adst2)


def _final_tc(o_ref, ah_ref, a4_ref, xh_ref, bg_ref, out_ref):
    a4 = a4_ref[...]
    t0 = a4[:, 0:1] + a4[:, 2:3]
    t1 = a4[:, 1:2] + a4[:, 3:4]
    aL0 = jnp.exp(jnp.where(t0 >= 0, t0, 0.2 * t0))
    aL1 = jnp.exp(jnp.where(t1 >= 0, t1, 0.2 * t1))
    den0 = jnp.sum(ah_ref[0], axis=0) + aL0 + 1e-16
    den1 = jnp.sum(ah_ref[1], axis=0) + aL1 + 1e-16
    num0 = o_ref[0] + aL0 * xh_ref[0]
    num1 = o_ref[1] + aL1 * xh_ref[1]
    out_ref[...] = 0.5 * (num0 / den0 + num1 / den1) + bg_ref[...]


def _final_kernel(o, ah, a4, xh, b_gat):
    return pl.pallas_call(
        _final_tc,
        grid=(NBLK,),
        in_specs=[
            pl.BlockSpec((NC, 128, OUT), lambda i: (0, i, 0)),
            pl.BlockSpec((NC, NS, 128, 1), lambda i: (0, 0, i, 0)),
            pl.BlockSpec((128, 4), lambda i: (i, 0)),
            pl.BlockSpec((NC, 128, OUT), lambda i: (0, i, 0)),
            pl.BlockSpec((1, OUT), lambda i: (0, 0)),
        ],
        out_specs=pl.BlockSpec((128, OUT), lambda i: (i, 0)),
        out_shape=jax.ShapeDtypeStruct((NPAD, OUT), jnp.float32),
    )(o, ah.reshape(NC, NS, NPAD, 1), a4, xh, b_gat.reshape(1, OUT))


def kernel(x, edge_index, edge_attr, W_sg, b_sg, W_gat, att_src, att_dst,
           b_gat):
    row = edge_index[0]
    col = edge_index[1]
    xpad = jnp.pad(x, ((0, NPAD - N), (0, 0)))

    hists = _deg_kernel(col, edge_attr)
    dinv, dsq, g = _prep_kernel(hists, xpad)
    p1 = _hop_kernel(g, row, col, edge_attr)
    h1, g1 = _combine_kernel(p1, dinv, dsq, xpad)
    p2 = _hop_kernel(g1, row, col, edge_attr)

    xh, a4 = _mm_kernel(p2, dinv, dsq, h1, W_sg, b_sg, W_gat, att_src,
                        att_dst)
    asrc2 = jnp.transpose(a4[:, 0:2])
    adst2 = jnp.transpose(a4[:, 2:4])
    o, ah = _gat_kernel(xh.reshape(NC * NPAD, OUT), row, col, asrc2, adst2)
    out = _final_kernel(o, ah, a4, xh, b_gat)
    return out[:N]


# speedup vs baseline: 46.5336x; 1.1134x over previous
import jax
import jax.numpy as jnp
from jax import lax
from jax.experimental import pallas as pl
from jax.experimental.pallas import tpu as pltpu
from jax.experimental.pallas import tpu_sc as plsc

N = 10000
E = 320000
D = 128
HID = 64
OUT = 128
NH = 2

NC, NS, L = 2, 16, 16
NW = NC * NS
NPAD = 10112
NBLK = NPAD // 128

_SELU_L = 1.0507009873554805
_SELU_A = 1.6732632423543772


def _mesh():
    return plsc.VectorSubcoreMesh(core_axis_name="c", subcore_axis_name="s",
                                  num_cores=NC, num_subcores=NS)


def _splat(v, j):
    return lax.gather(
        v, jnp.full((L, 1), j, jnp.int32),
        dimension_numbers=lax.GatherDimensionNumbers(
            offset_dims=(), collapsed_slice_dims=(0,), start_index_map=(0,)),
        slice_sizes=(1,), mode=lax.GatherScatterMode.PROMISE_IN_BOUNDS)


def _zero_ref_1d(ref, n):
    z = jnp.zeros((L,), jnp.float32)

    def body(i, _):
        ref[pl.ds(i * L, L)] = z
        return 0

    lax.fori_loop(0, n // L, body, 0)


def _zero_rows(ref, nrows, width):
    z = jnp.zeros((L,), jnp.float32)

    def body(i, _):
        for k in range(width // L):
            ref[i, pl.ds(k * L, L)] = z
        return 0

    lax.fori_loop(0, nrows, body, 0)


def _acc_zero_and_writeback(rows, acc, dst, r0, zero_phase):
    if zero_phase:
        for k in range(4):
            pltpu.sync_copy(rows.at[pl.ds(0, 128)],
                            acc.at[pl.ds(r0 + k * 128, 128)])
        pltpu.sync_copy(rows.at[pl.ds(0, 120)], acc.at[pl.ds(r0 + 512, 120)])
    else:
        for k in range(4):
            pltpu.sync_copy(acc.at[pl.ds(r0 + k * 128, 128)],
                            dst.at[pl.ds(r0 + k * 128, 128)])
        pltpu.sync_copy(acc.at[pl.ds(r0 + 512, 120)],
                        dst.at[pl.ds(r0 + 512, 120)])


def _deg_sc(col_hbm, ew_hbm, out_hbm, colbuf, ewbuf, hist):
    c = lax.axis_index("c")
    s = lax.axis_index("s")
    wid = c * NS + s
    _zero_ref_1d(hist, NPAD)
    eper = E // NW
    chunk = 2000

    def chunk_body(t, _):
        base = wid * eper + t * chunk
        pltpu.sync_copy(col_hbm.at[pl.ds(base, chunk)], colbuf)
        pltpu.sync_copy(ew_hbm.at[pl.ds(base, chunk)], ewbuf)

        def grp(g, _):
            cv = colbuf[pl.ds(g * L, L)]
            ev = ewbuf[pl.ds(g * L, L)]
            plsc.addupdate_scatter(hist, [cv], ev)
            return 0

        lax.fori_loop(0, chunk // L, grp, 0)
        return 0

    lax.fori_loop(0, eper // chunk, chunk_body, 0)
    pltpu.sync_copy(hist, out_hbm.at[wid])


def _deg_kernel(col, ew):
    return pl.kernel(
        _deg_sc,
        out_type=jax.ShapeDtypeStruct((NW, NPAD), jnp.float32),
        mesh=_mesh(),
        compiler_params=pltpu.CompilerParams(needs_layout_passes=False),
        scratch_types=[
            pltpu.VMEM((2000,), jnp.int32),
            pltpu.VMEM((2000,), jnp.float32),
            pltpu.VMEM((NPAD,), jnp.float32),
        ],
    )(col, ew)


def _prep_tc(hist_ref, x_ref, dinv_ref, dsq_ref, g_ref):
    deg = jnp.sum(hist_ref[...], axis=0) + 1.0
    dinv = lax.rsqrt(deg)
    dinv_ref[...] = dinv
    dsq_ref[...] = 1.0 / deg
    g_ref[...] = x_ref[...] * dinv


def _prep_kernel(hists, xpad):
    return pl.pallas_call(
        _prep_tc,
        grid=(NBLK,),
        in_specs=[
            pl.BlockSpec((NW, 128, 1), lambda i: (0, i, 0)),
            pl.BlockSpec((128, D), lambda i: (i, 0)),
        ],
        out_specs=[
            pl.BlockSpec((128, 1), lambda i: (i, 0)),
            pl.BlockSpec((128, 1), lambda i: (i, 0)),
            pl.BlockSpec((128, D), lambda i: (i, 0)),
        ],
        out_shape=[
            jax.ShapeDtypeStruct((NPAD, 1), jnp.float32),
            jax.ShapeDtypeStruct((NPAD, 1), jnp.float32),
            jax.ShapeDtypeStruct((NPAD, D), jnp.float32),
        ],
    )(hists.reshape(NW, NPAD, 1), xpad)


def _scale_rows(rows, wbuf, nrows):
    @plsc.parallel_loop(0, nrows // L)
    def _(g):
        wv = wbuf[pl.ds(g * L, L)]
        for j in range(L):
            cj = _splat(wv, j)
            for k in range(D // L):
                sl = pl.ds(k * L, L)
                rows[g * L + j, sl] = rows[g * L + j, sl] * cj


def _hop_sc(g_hbm, row_hbm, col_hbm, ew_hbm, out_hbm,
            rb0, rb1, cb0, cb1, eb0, eb1, rows0, rows1,
            rbE, cbE, ebE,
            acc, sg0, sg1, ss0, ss1, si0, si1):
    c = lax.axis_index("c")
    s = lax.axis_index("s")
    wid = c * NS + s
    eper = E // NW
    r0 = s * (NPAD // NS)
    CH = 160
    NCHUNK = 62
    ebase = wid * eper

    _zero_rows(rows0, 128, D)
    _acc_zero_and_writeback(rows0, acc, None, r0, True)
    plsc.subcore_barrier()

    rb = (rb0, rb1)
    cb = (cb0, cb1)
    eb = (eb0, eb1)
    rw = (rows0, rows1)
    sg = (sg0, sg1)
    ss = (ss0, ss1)
    si = (si0, si1)

    def idx_issue(t, b):
        base = ebase + t * CH
        pltpu.async_copy(row_hbm.at[pl.ds(base, CH)], rb[b], si[b])
        pltpu.async_copy(col_hbm.at[pl.ds(base, CH)], cb[b], si[b])
        pltpu.async_copy(ew_hbm.at[pl.ds(base, CH)], eb[b], si[b])

    def idx_wait(t, b):
        base = ebase + t * CH
        pltpu.make_async_copy(row_hbm.at[pl.ds(base, CH)], rb[b], si[b]).wait()
        pltpu.make_async_copy(col_hbm.at[pl.ds(base, CH)], cb[b], si[b]).wait()
        pltpu.make_async_copy(ew_hbm.at[pl.ds(base, CH)], eb[b], si[b]).wait()

    def gather_issue(b):
        pltpu.async_copy(g_hbm.at[rb[b]], rw[b], sg[b])

    def gather_wait(b):
        pltpu.make_async_copy(g_hbm.at[rb[b]], rw[b], sg[b]).wait()

    def scatter_issue(b):
        pltpu.async_copy(rw[b], acc.at[cb[b]], ss[b], add=True)

    def scatter_wait(b):
        pltpu.make_async_copy(rw[b], acc.at[cb[b]], ss[b]).wait()

    pltpu.sync_copy(row_hbm.at[pl.ds(ebase, CH)], rb0)
    pltpu.sync_copy(col_hbm.at[pl.ds(ebase, CH)], cb0)
    pltpu.sync_copy(ew_hbm.at[pl.ds(ebase, CH)], eb0)
    gather_issue(0)

    def pair(k, _):
        for b in (0, 1):
            t = k * 2 + b
            nb = 1 - b

            @pl.when(t > 0)
            def _():
                scatter_wait(nb)

            @pl.when(t < NCHUNK - 1)
            def _():
                idx_issue(t + 1, nb)
                idx_wait(t + 1, nb)
                gather_issue(nb)

            gather_wait(b)
            _scale_rows(rw[b], eb[b], CH)
            scatter_issue(b)
        return 0

    lax.fori_loop(0, NCHUNK // 2, pair, 0)
    scatter_wait(1)

    base = ebase + NCHUNK * CH
    pltpu.sync_copy(row_hbm.at[pl.ds(base, 80)], rbE)
    pltpu.sync_copy(col_hbm.at[pl.ds(base, 80)], cbE)
    pltpu.sync_copy(ew_hbm.at[pl.ds(base, 80)], ebE)
    pltpu.async_copy(g_hbm.at[rbE], rows0.at[pl.ds(0, 80)], sg0).wait()
    _scale_rows(rows0, ebE, 80)
    pltpu.sync_copy(rows0.at[pl.ds(0, 80)], acc.at[cbE], add=True)

    plsc.subcore_barrier()
    _acc_zero_and_writeback(rows0, acc, out_hbm.at[c], r0, False)


def _hop_kernel(g, row, col, ew):
    return pl.kernel(
        _hop_sc,
        out_type=jax.ShapeDtypeStruct((NC, NPAD, D), jnp.float32),
        mesh=_mesh(),
        compiler_params=pltpu.CompilerParams(needs_layout_passes=False),
        scratch_types=[
            pltpu.VMEM((160,), jnp.int32),
            pltpu.VMEM((160,), jnp.int32),
            pltpu.VMEM((160,), jnp.int32),
            pltpu.VMEM((160,), jnp.int32),
            pltpu.VMEM((160,), jnp.float32),
            pltpu.VMEM((160,), jnp.float32),
            pltpu.VMEM((160, D), jnp.float32),
            pltpu.VMEM((160, D), jnp.float32),
            pltpu.VMEM((80,), jnp.int32),
            pltpu.VMEM((80,), jnp.int32),
            pltpu.VMEM((80,), jnp.float32),
            pltpu.VMEM_SHARED((NPAD, D), jnp.float32),
            pltpu.SemaphoreType.DMA,
            pltpu.SemaphoreType.DMA,
            pltpu.SemaphoreType.DMA,
            pltpu.SemaphoreType.DMA,
            pltpu.SemaphoreType.DMA,
            pltpu.SemaphoreType.DMA,
        ],
    )(g, row, col, ew)


def _combine_tc(p_ref, dinv_ref, dsq_ref, h_ref, hn_ref, gn_ref):
    sm = p_ref[0] + p_ref[1]
    hn = dinv_ref[...] * sm + dsq_ref[...] * h_ref[...]
    hn_ref[...] = hn
    gn_ref[...] = dinv_ref[...] * hn


def _combine_kernel(p, dinv, dsq, h):
    return pl.pallas_call(
        _combine_tc,
        grid=(NBLK,),
        in_specs=[
            pl.BlockSpec((NC, 128, D), lambda i: (0, i, 0)),
            pl.BlockSpec((128, 1), lambda i: (i, 0)),
            pl.BlockSpec((128, 1), lambda i: (i, 0)),
            pl.BlockSpec((128, D), lambda i: (i, 0)),
        ],
        out_specs=[
            pl.BlockSpec((128, D), lambda i: (i, 0)),
            pl.BlockSpec((128, D), lambda i: (i, 0)),
        ],
        out_shape=[
            jax.ShapeDtypeStruct((NPAD, D), jnp.float32),
            jax.ShapeDtypeStruct((NPAD, D), jnp.float32),
        ],
    )(p, dinv, dsq, h)


def _mm_tc(p_ref, dinv_ref, dsq_ref, h1_ref, wsg_ref, bsg_ref, wgat_ref,
           asrc_ref, adst_ref, xh_ref, a4_ref):
    h2 = dinv_ref[...] * (p_ref[0] + p_ref[1]) + dsq_ref[...] * h1_ref[...]
    t = jnp.dot(h2, wsg_ref[...],
                preferred_element_type=jnp.float32) + bsg_ref[...]
    t = _SELU_L * jnp.where(t > 0, t, _SELU_A * (jnp.exp(t) - 1.0))
    y = jnp.dot(t, wgat_ref[...], preferred_element_type=jnp.float32)
    y0 = y[:, :OUT]
    y1 = y[:, OUT:]
    xh_ref[0] = y0
    xh_ref[1] = y1
    a4_ref[...] = jnp.concatenate([
        jnp.sum(y0 * asrc_ref[0:1, :], axis=1, keepdims=True),
        jnp.sum(y1 * asrc_ref[1:2, :], axis=1, keepdims=True),
        jnp.sum(y0 * adst_ref[0:1, :], axis=1, keepdims=True),
        jnp.sum(y1 * adst_ref[1:2, :], axis=1, keepdims=True),
    ], axis=1)


def _mm_kernel(p2, dinv, dsq, h1, W_sg, b_sg, W_gat, att_src, att_dst):
    return pl.pallas_call(
        _mm_tc,
        grid=(NBLK,),
        in_specs=[
            pl.BlockSpec((NC, 128, D), lambda i: (0, i, 0)),
            pl.BlockSpec((128, 1), lambda i: (i, 0)),
            pl.BlockSpec((128, 1), lambda i: (i, 0)),
            pl.BlockSpec((128, D), lambda i: (i, 0)),
            pl.BlockSpec((D, HID), lambda i: (0, 0)),
            pl.BlockSpec((1, HID), lambda i: (0, 0)),
            pl.BlockSpec((HID, NH * OUT), lambda i: (0, 0)),
            pl.BlockSpec((NH, OUT), lambda i: (0, 0)),
            pl.BlockSpec((NH, OUT), lambda i: (0, 0)),
        ],
        out_specs=[
            pl.BlockSpec((NH, 128, OUT), lambda i: (0, i, 0)),
            pl.BlockSpec((128, 4), lambda i: (i, 0)),
        ],
        out_shape=[
            jax.ShapeDtypeStruct((NH, NPAD, OUT), jnp.float32),
            jax.ShapeDtypeStruct((NPAD, 4), jnp.float32),
        ],
    )(p2, dinv, dsq, h1, W_sg, b_sg.reshape(1, HID), W_gat, att_src, att_dst)


def _alpha_sc(row_hbm, col_hbm, asrc_hbm, adst_hbm, pk_hbm, ah_hbm,
              rowbuf, colbuf, asrcv, adstv, hist, pbuf):
    c = lax.axis_index("c")
    s = lax.axis_index("s")
    eper = E // NS
    cbase = c * NPAD
    chunk = 2000
    G = chunk // L

    pltpu.sync_copy(asrc_hbm.at[c], asrcv)
    pltpu.sync_copy(adst_hbm.at[c], adstv)
    _zero_ref_1d(hist, NPAD)

    def chunk_body(t, _):
        base = s * eper + t * chunk
        pltpu.sync_copy(row_hbm.at[pl.ds(base, chunk)], rowbuf)
        pltpu.sync_copy(col_hbm.at[pl.ds(base, chunk)], colbuf)

        def grp(g, _):
            sl = pl.ds(g * L, L)
            rv = rowbuf[sl]
            cv = colbuf[sl]
            sv = plsc.load_gather(asrcv, [rv])
            dv = plsc.load_gather(adstv, [cv])
            tt = sv + dv
            tt = jnp.where(tt >= 0, tt, 0.2 * tt)
            al = jnp.exp(tt)
            plsc.addupdate_scatter(hist, [cv], al)
            pbuf[pl.ds(g * 3 * L, L)] = rv + cbase
            pbuf[pl.ds(g * 3 * L + L, L)] = cv
            pbuf[pl.ds(g * 3 * L + 2 * L, L)] = plsc.bitcast(al, jnp.int32)
            return 0

        lax.fori_loop(0, G, grp, 0)
        pltpu.sync_copy(
            pbuf,
            pk_hbm.at[pl.ds(c * E * 3 + (s * eper + t * chunk) * 3, G * 3 * L)])
        return 0

    lax.fori_loop(0, eper // chunk, chunk_body, 0)
    pltpu.sync_copy(hist, ah_hbm.at[c, s])


def _alpha_kernel(row, col, asrc2, adst2):
    return pl.kernel(
        _alpha_sc,
        out_type=[
            jax.ShapeDtypeStruct((NC * E * 3,), jnp.int32),
            jax.ShapeDtypeStruct((NC, NS, NPAD), jnp.float32),
        ],
        mesh=_mesh(),
        compiler_params=pltpu.CompilerParams(needs_layout_passes=False),
        scratch_types=[
            pltpu.VMEM((2000,), jnp.int32),
            pltpu.VMEM((2000,), jnp.int32),
            pltpu.VMEM((NPAD,), jnp.float32),
            pltpu.VMEM((NPAD,), jnp.float32),
            pltpu.VMEM((NPAD,), jnp.float32),
            pltpu.VMEM((6000,), jnp.int32),
        ],
    )(row, col, asrc2, adst2)


def _scale_rows_pk(rows, pkb, nrows):
    @plsc.parallel_loop(0, nrows // L)
    def _(g):
        wv = plsc.bitcast(pkb[pl.ds(g * 3 * L + 2 * L, L)], jnp.float32)
        for j in range(L):
            cj = _splat(wv, j)
            for k in range(OUT // L):
                sl = pl.ds(k * L, L)
                rows[g * L + j, sl] = rows[g * L + j, sl] * cj


def _gsc_sc(xh_hbm, pk_hbm, o_hbm,
            pk0, pk1, rb0, rb1, cb0, cb1, rows0, rows1,
            acc, sg0, sg1, ss0, ss1, si0, si1):
    c = lax.axis_index("c")
    s = lax.axis_index("s")
    r0 = s * (NPAD // NS)
    CH = 160
    GR = CH // L
    NCHUNK = 125
    gbase = c * E * 3 + s * (E // NS) * 3

    _zero_rows(rows0, 128, OUT)
    _acc_zero_and_writeback(rows0, acc, None, r0, True)
    plsc.subcore_barrier()

    pk = (pk0, pk1)
    rb = (rb0, rb1)
    cb = (cb0, cb1)
    rw = (rows0, rows1)
    sg = (sg0, sg1)
    ss = (ss0, ss1)
    si = (si0, si1)

    def idx_issue(t, b):
        pltpu.async_copy(pk_hbm.at[pl.ds(gbase + t * CH * 3, CH * 3)],
                         pk[b], si[b])

    def idx_wait(t, b):
        pltpu.make_async_copy(pk_hbm.at[pl.ds(gbase + t * CH * 3, CH * 3)],
                              pk[b], si[b]).wait()

    def unpack(b):
        @plsc.parallel_loop(0, GR)
        def _(g):
            sl = pl.ds(g * L, L)
            rb[b][sl] = pk[b][pl.ds(g * 3 * L, L)]
            cb[b][sl] = pk[b][pl.ds(g * 3 * L + L, L)]

    def gather_issue(b):
        pltpu.async_copy(xh_hbm.at[rb[b]], rw[b], sg[b])

    def gather_wait(b):
        pltpu.make_async_copy(xh_hbm.at[rb[b]], rw[b], sg[b]).wait()

    def scatter_issue(b):
        pltpu.async_copy(rw[b], acc.at[cb[b]], ss[b], add=True)

    def scatter_wait(b):
        pltpu.make_async_copy(rw[b], acc.at[cb[b]], ss[b]).wait()

    pltpu.sync_copy(pk_hbm.at[pl.ds(gbase, CH * 3)], pk0)
    unpack(0)
    gather_issue(0)

    def pair(k, _):
        for b in (0, 1):
            t = k * 2 + b
            nb = 1 - b

            @pl.when(t > 0)
            def _():
                scatter_wait(nb)

            idx_issue(t + 1, nb)
            idx_wait(t + 1, nb)
            unpack(nb)
            gather_issue(nb)

            gather_wait(b)
            _scale_rows_pk(rw[b], pk[b], CH)
            scatter_issue(b)
        return 0

    lax.fori_loop(0, (NCHUNK - 1) // 2, pair, 0)

    scatter_wait(1)
    gather_wait(0)
    _scale_rows_pk(rows0, pk0, CH)
    pltpu.sync_copy(rows0, acc.at[cb0], add=True)

    plsc.subcore_barrier()
    _acc_zero_and_writeback(rows0, acc, o_hbm.at[c], r0, False)


def _gat_kernel(xhflat, pk):
    return pl.kernel(
        _gsc_sc,
        out_type=jax.ShapeDtypeStruct((NC, NPAD, OUT), jnp.float32),
        mesh=_mesh(),
        compiler_params=pltpu.CompilerParams(needs_layout_passes=False),
        scratch_types=[
            pltpu.VMEM((480,), jnp.int32),
            pltpu.VMEM((480,), jnp.int32),
            pltpu.VMEM((160,), jnp.int32),
            pltpu.VMEM((160,), jnp.int32),
            pltpu.VMEM((160,), jnp.int32),
            pltpu.VMEM((160,), jnp.int32),
            pltpu.VMEM((160, OUT), jnp.float32),
            pltpu.VMEM((160, OUT), jnp.float32),
            pltpu.VMEM_SHARED((NPAD, OUT), jnp.float32),
            pltpu.SemaphoreType.DMA,
            pltpu.SemaphoreType.DMA,
            pltpu.SemaphoreType.DMA,
            pltpu.SemaphoreType.DMA,
            pltpu.SemaphoreType.DMA,
            pltpu.SemaphoreType.DMA,
        ],
    )(xhflat, pk)


def _final_tc(o_ref, ah_ref, a4_ref, xh_ref, bg_ref, out_ref):
    a4 = a4_ref[...]
    t0 = a4[:, 0:1] + a4[:, 2:3]
    t1 = a4[:, 1:2] + a4[:, 3:4]
    aL0 = jnp.exp(jnp.where(t0 >= 0, t0, 0.2 * t0))
    aL1 = jnp.exp(jnp.where(t1 >= 0, t1, 0.2 * t1))
    den0 = jnp.sum(ah_ref[0], axis=0) + aL0 + 1e-16
    den1 = jnp.sum(ah_ref[1], axis=0) + aL1 + 1e-16
    num0 = o_ref[0] + aL0 * xh_ref[0]
    num1 = o_ref[1] + aL1 * xh_ref[1]
    out_ref[...] = 0.5 * (num0 / den0 + num1 / den1) + bg_ref[...]


def _final_kernel(o, ah, a4, xh, b_gat):
    return pl.pallas_call(
        _final_tc,
        grid=(NBLK,),
        in_specs=[
            pl.BlockSpec((NC, 128, OUT), lambda i: (0, i, 0)),
            pl.BlockSpec((NC, NS, 128, 1), lambda i: (0, 0, i, 0)),
            pl.BlockSpec((128, 4), lambda i: (i, 0)),
            pl.BlockSpec((NC, 128, OUT), lambda i: (0, i, 0)),
            pl.BlockSpec((1, OUT), lambda i: (0, 0)),
        ],
        out_specs=pl.BlockSpec((128, OUT), lambda i: (i, 0)),
        out_shape=jax.ShapeDtypeStruct((NPAD, OUT), jnp.float32),
    )(o, ah.reshape(NC, NS, NPAD, 1), a4, xh, b_gat.reshape(1, OUT))


def kernel(x, edge_index, edge_attr, W_sg, b_sg, W_gat, att_src, att_dst,
           b_gat):
    row = edge_index[0]
    col = edge_index[1]
    xpad = jnp.pad(x, ((0, NPAD - N), (0, 0)))

    hists = _deg_kernel(col, edge_attr)
    dinv, dsq, g = _prep_kernel(hists, xpad)
    p1 = _hop_kernel(g, row, col, edge_attr)
    h1, g1 = _combine_kernel(p1, dinv, dsq, xpad)
    p2 = _hop_kernel(g1, row, col, edge_attr)

    xh, a4 = _mm_kernel(p2, dinv, dsq, h1, W_sg, b_sg, W_gat, att_src,
                        att_dst)
    asrc2 = jnp.transpose(a4[:, 0:2])
    adst2 = jnp.transpose(a4[:, 2:4])
    pk, ah = _alpha_kernel(row, col, asrc2, adst2)
    o = _gat_kernel(xh.reshape(NC * NPAD, OUT), pk)
    out = _final_kernel(o, ah, a4, xh, b_gat)
    return out[:N]
